# Initial kernel scaffold; baseline (speedup 1.0000x reference)
import jax
import jax.numpy as jnp
from jax.experimental import pallas as pl


def kernel(t, pos, idcs_airfoil, velocity_in, wall_distance, surface_frame, knn_indices, params):
    raise NotImplementedError("write your pallas kernel here")


# speedup vs baseline: 3.5104x; 3.5104x over previous
import functools
import math

import jax
import jax.numpy as jnp
from jax import lax
from jax.experimental import pallas as pl
from jax.experimental.pallas import tpu as pltpu

B = 1
N = 8192
K = 16
D = 128
MODES = 256
HEADS = 8
DH = D // HEADS
TIN = 5
TOUT = 5
NF = 16
TF = 2 * NF + 1
TILE = 256
NT = N // TILE


def _ln(x, g, b):
    m = x.mean(-1, keepdims=True)
    v = ((x - m) ** 2).mean(-1, keepdims=True)
    return (x - m) * lax.rsqrt(v + 1e-5) * g + b


def _gelu(x):
    c = math.sqrt(2.0 / math.pi)
    return 0.5 * x * (1.0 + jnp.tanh(c * (x + 0.044715 * (x ** 3))))


def _full(spec_shape):
    nd = len(spec_shape)
    return pl.BlockSpec(spec_shape, lambda i, _nd=nd: (0,) * _nd)


def _enc_body(f_ref, w1, b1, w2, b2, w3, b3, o_ref):
    x = _gelu(jnp.dot(f_ref[...], w1[...], preferred_element_type=jnp.float32) + b1[...])
    x = _gelu(jnp.dot(x, w2[...], preferred_element_type=jnp.float32) + b2[...])
    o_ref[...] = jnp.dot(x, w3[...], preferred_element_type=jnp.float32) + b3[...]


def _encoder(feats, wp):
    R = feats.shape[0]
    w1, b1, w2, b2, w3, b3 = wp
    return pl.pallas_call(
        _enc_body,
        grid=(R // TILE,),
        in_specs=[
            pl.BlockSpec((TILE, feats.shape[1]), lambda i: (i, 0)),
            _full(w1.shape), _full(b1.shape), _full(w2.shape),
            _full(b2.shape), _full(w3.shape), _full(b3.shape),
        ],
        out_specs=pl.BlockSpec((TILE, D), lambda i: (i, 0)),
        out_shape=jax.ShapeDtypeStruct((R, D), jnp.float32),
    )(feats, w1, b1, w2, b2, w3, b3)


def _stage_a_body(x_ref, pp_ref, g1, bb1, wh, h_ref, g_ref):
    h = _ln(x_ref[...], g1[...], bb1[...])
    h_ref[...] = h
    g_ref[...] = jnp.dot(h, wh[...], preferred_element_type=jnp.float32) + pp_ref[...]


def _stage_a(x, pp, bp):
    R = x.shape[0]
    return pl.pallas_call(
        _stage_a_body,
        grid=(R // TILE,),
        in_specs=[
            pl.BlockSpec((TILE, D), lambda i: (i, 0)),
            pl.BlockSpec((TILE, D), lambda i: (i % NT, 0)),
            _full((D,)), _full((D,)), _full((D, D)),
        ],
        out_specs=[
            pl.BlockSpec((TILE, D), lambda i: (i, 0)),
            pl.BlockSpec((TILE, D), lambda i: (i, 0)),
        ],
        out_shape=[
            jax.ShapeDtypeStruct((R, D), jnp.float32),
            jax.ShapeDtypeStruct((R, D), jnp.float32),
        ],
    )(x, pp, bp['ln1g'], bp['ln1b'], bp['wh_t'])


def _stage_b_body(gath_ref, x_ref, h_ref, pp_ref,
                  bmsg, w1, w2, bupd, g2, bb2, wqkbd, vbd, den, exp_m,
                  g3, bb3, wf1, bf1, wf2, bf2, o_ref):
    gath = gath_ref[...].reshape(TILE, K, D)
    c = pp_ref[...] - bmsg[...]
    msg = _gelu(gath - c[:, None, :])
    agg = jnp.mean(msg, axis=1)
    x1 = (x_ref[...]
          + jnp.dot(h_ref[...], w1[...], preferred_element_type=jnp.float32)
          + jnp.dot(agg, w2[...], preferred_element_type=jnp.float32)
          + bupd[...])
    h2 = _ln(x1, g2[...], bb2[...])
    s = jnp.dot(h2, wqkbd[...], preferred_element_type=jnp.float32)
    e = jnp.exp(s)
    denom = jnp.dot(e, den[...], preferred_element_type=jnp.float32)
    dfull = jnp.dot(denom, exp_m[...], preferred_element_type=jnp.float32)
    numer = jnp.dot(e, vbd[...], preferred_element_type=jnp.float32)
    x2 = x1 + numer / dfull
    h3 = _ln(x2, g3[...], bb3[...])
    f = _gelu(jnp.dot(h3, wf1[...], preferred_element_type=jnp.float32) + bf1[...])
    o_ref[...] = x2 + jnp.dot(f, wf2[...], preferred_element_type=jnp.float32) + bf2[...]


def _stage_b(gath, x, h, pp, bp):
    R = x.shape[0]
    return pl.pallas_call(
        _stage_b_body,
        grid=(R // TILE,),
        in_specs=[
            pl.BlockSpec((TILE * K, D), lambda i: (i, 0)),
            pl.BlockSpec((TILE, D), lambda i: (i, 0)),
            pl.BlockSpec((TILE, D), lambda i: (i, 0)),
            pl.BlockSpec((TILE, D), lambda i: (i % NT, 0)),
            _full((D,)), _full((D, D)), _full((D, D)), _full((D,)),
            _full((D,)), _full((D,)),
            _full((D, HEADS * MODES)), _full((HEADS * MODES, D)),
            _full((HEADS * MODES, HEADS)), _full((HEADS, D)),
            _full((D,)), _full((D,)),
            _full((D, 2 * D)), _full((2 * D,)), _full((2 * D, D)), _full((D,)),
        ],
        out_specs=pl.BlockSpec((TILE, D), lambda i: (i, 0)),
        out_shape=jax.ShapeDtypeStruct((R, D), jnp.float32),
    )(gath, x, h, pp,
      bp['bmsg'], bp['w1_t'], bp['w2_t'], bp['bupd'],
      bp['ln2g'], bp['ln2b'], bp['wqkbd'], bp['vbd'], bp['den'], bp['exp_m'],
      bp['ln3g'], bp['ln3b'], bp['wf1_t'], bp['bf1'], bp['wf2_t'], bp['bf2'])


def _gather_rows(g, idx_flat):
    return jnp.take(g, idx_flat, axis=0)


def _gru_body(lat_ref, tip_ref, gin, bin_, w_ih0, w_hh0, bih0, bhh0,
              w_ih1, w_hh1, bih1, bhh1, gout, bout,
              pw1, pb1, pw2, pb2, pw3, pb3, ftp_ref, o_ref):
    xs = [_ln(lat_ref[t] + tip_ref[t], gin[...], bin_[...]) for t in range(TIN)]
    for (wi, wh, bi, bh) in ((w_ih0, w_hh0, bih0, bhh0), (w_ih1, w_hh1, bih1, bhh1)):
        h = jnp.zeros((TILE, D), jnp.float32)
        ys = []
        for t in range(TIN):
            gi = jnp.dot(xs[t], wi[...], preferred_element_type=jnp.float32) + bi[...]
            gh = jnp.dot(h, wh[...], preferred_element_type=jnp.float32) + bh[...]
            r = jax.nn.sigmoid(gi[:, :D] + gh[:, :D])
            z = jax.nn.sigmoid(gi[:, D:2 * D] + gh[:, D:2 * D])
            nn = jnp.tanh(gi[:, 2 * D:] + r * gh[:, 2 * D:])
            h = (1.0 - z) * nn + z * h
            ys.append(h)
        xs = ys
    seq = [_ln(y, gout[...], bout[...]) for y in xs]
    flat = jnp.concatenate(seq, axis=-1)
    p = _gelu(jnp.dot(flat, pw1[...], preferred_element_type=jnp.float32) + pb1[...])
    p = _gelu(jnp.dot(p, pw2[...], preferred_element_type=jnp.float32) + pb2[...])
    p = jnp.dot(p, pw3[...], preferred_element_type=jnp.float32) + pb3[...]
    for s in range(TOUT):
        o_ref[s] = p[:, s * D:(s + 1) * D] + ftp_ref[s]


def _temporal(lat, tip, ftp, wp):
    args = [lat, tip] + list(wp) + [ftp]
    in_specs = [pl.BlockSpec((TIN, TILE, D), lambda i: (0, i, 0)),
                _full((TIN, D))]
    in_specs += [_full(w.shape) for w in wp]
    in_specs += [_full((TOUT, D))]
    return pl.pallas_call(
        _gru_body,
        grid=(NT,),
        in_specs=in_specs,
        out_specs=pl.BlockSpec((TOUT, TILE, D), lambda i: (0, i, 0)),
        out_shape=jax.ShapeDtypeStruct((TOUT, N, D), jnp.float32),
    )(*args)


def _dec_body(x_ref, v0_ref, w1, b1, w2, b2, w3, b3, o_ref):
    acc = v0_ref[...]
    for s in range(TOUT):
        y = _gelu(jnp.dot(x_ref[s], w1[...], preferred_element_type=jnp.float32) + b1[...])
        y = _gelu(jnp.dot(y, w2[...], preferred_element_type=jnp.float32) + b2[...])
        y = jnp.dot(y, w3[...], preferred_element_type=jnp.float32) + b3[...]
        acc = acc + y
        o_ref[s] = acc


def _decoder(x, v0, wp):
    w1, b1, w2, b2, w3, b3 = wp
    return pl.pallas_call(
        _dec_body,
        grid=(NT,),
        in_specs=[
            pl.BlockSpec((TOUT, TILE, D), lambda i: (0, i, 0)),
            pl.BlockSpec((TILE, 3), lambda i: (i, 0)),
            _full(w1.shape), _full(b1.shape), _full(w2.shape),
            _full(b2.shape), _full(w3.shape), _full(b3.shape),
        ],
        out_specs=pl.BlockSpec((TOUT, TILE, 3), lambda i: (0, i, 0)),
        out_shape=jax.ShapeDtypeStruct((TOUT, N, 3), jnp.float32),
    )(x, v0, w1, b1, w2, b2, w3, b3)


def _pp_body(pos_ref, wr_ref, o_ref):
    o_ref[...] = jnp.dot(pos_ref[...], wr_ref[...], preferred_element_type=jnp.float32)


def _pos_proj(pos, wr_all):
    nb = wr_all.shape[1]
    return pl.pallas_call(
        _pp_body,
        grid=(NT,),
        in_specs=[pl.BlockSpec((TILE, 3), lambda i: (i, 0)), _full((3, nb))],
        out_specs=pl.BlockSpec((TILE, nb), lambda i: (i, 0)),
        out_shape=jax.ShapeDtypeStruct((N, nb), jnp.float32),
    )(pos, wr_all)


def _prep_block(p):
    wmsg = p['Wmsg']
    basis = p['basis']
    kb = basis.reshape(MODES, HEADS, DH)
    kbd = jax.scipy.linalg.block_diag(*[kb[:, h, :].T for h in range(HEADS)])
    vb = (basis @ p['Wv'].T).reshape(MODES, HEADS, DH)
    vbd = jax.scipy.linalg.block_diag(*[vb[:, h, :] for h in range(HEADS)])
    den = jnp.kron(jnp.eye(HEADS, dtype=jnp.float32), jnp.ones((MODES, 1), jnp.float32))
    exp_m = jnp.kron(jnp.eye(HEADS, dtype=jnp.float32), jnp.ones((1, DH), jnp.float32))
    (f1w, f1b), (f2w, f2b) = p['ffn']
    return {
        'ln1g': p['ln1'][0], 'ln1b': p['ln1'][1],
        'ln2g': p['ln2'][0], 'ln2b': p['ln2'][1],
        'ln3g': p['ln3'][0], 'ln3b': p['ln3'][1],
        'wh_t': wmsg[:, :D].T, 'wr_t': wmsg[:, D:].T, 'bmsg': p['bmsg'],
        'w1_t': p['Wupd'][:, :D].T, 'w2_t': p['Wupd'][:, D:].T, 'bupd': p['bupd'],
        'wqkbd': (p['Wq'].T @ kbd) / math.sqrt(DH), 'vbd': vbd,
        'den': den, 'exp_m': exp_m,
        'wf1_t': f1w.T, 'bf1': f1b, 'wf2_t': f2w.T, 'bf2': f2b,
    }


def _mlp_t(params):
    out = []
    for (w, b) in params:
        out.append(w.T)
        out.append(b)
    return out


def _mlp_jax(params, x):
    n = len(params)
    for i, (w, b) in enumerate(params):
        x = x @ w.T + b
        if i < n - 1:
            x = jax.nn.gelu(x)
    return x


def _fourier(t):
    freqs = jnp.pi * (2.0 ** jnp.arange(NF))
    a = t[..., None] * freqs
    return jnp.concatenate([t[..., None], jnp.sin(a), jnp.cos(a)], -1)


def _run_blocks(x, blocks, pps, knn_flat, R):
    for bp, pp in zip(blocks, pps):
        h, g = _stage_a(x, pp, bp)
        gath = _gather_rows(g, knn_flat)
        x = _stage_b(gath, x, h, pp, bp)
    return x


def kernel(t, pos, idcs_airfoil, velocity_in, wall_distance, surface_frame,
           knn_indices, params):
    temb = _fourier(t)
    in_emb = temb[:, :TIN]
    out_emb = temb[:, TIN:]
    tip = _mlp_jax(params['temporal_input_proj'], in_emb)[0]
    ftp = _mlp_jax(params['future_time_proj'], out_emb)[0]

    pos2 = pos[0]
    mask = jnp.zeros((N,), jnp.float32).at[idcs_airfoil[0]].set(1.0)
    wall = jnp.log1p(wall_distance[0])[:, None]
    sf = surface_frame[0]

    tfb = jnp.broadcast_to(in_emb[0][:, None, :], (TIN, N, TF))
    rest = jnp.concatenate([wall, mask[:, None], sf], -1)
    feats = jnp.concatenate([
        jnp.broadcast_to(pos2[None], (TIN, N, 3)),
        velocity_in[0],
        tfb,
        jnp.broadcast_to(rest[None], (TIN, N, 11)),
    ], -1).reshape(TIN * N, 3 + 3 + TF + 11)

    enc_blocks = [_prep_block(p) for p in params['encoder_blocks']]
    fc_blocks = [_prep_block(p) for p in params['forecast_blocks']]

    wr_all = jnp.concatenate([bp['wr_t'] for bp in enc_blocks + fc_blocks], axis=1)
    pp_all = _pos_proj(pos2, wr_all)
    pps = [pp_all[:, i * D:(i + 1) * D] for i in range(4)]

    knn = knn_indices[0].astype(jnp.int32)
    base = jnp.arange(TIN, dtype=jnp.int32)[:, None, None] * N
    knn_exp = (knn[None] + base).reshape(TIN * N * K)

    R = TIN * N

    x = _encoder(feats, _mlp_t(params['frame_encoder']))
    x = _run_blocks(x, enc_blocks, pps[:2], knn_exp, R)

    lat = x.reshape(TIN, N, D)
    gru = params['gru']
    wp = [params['temporal_input_norm'][0], params['temporal_input_norm'][1],
          gru[0]['Wih'].T, gru[0]['Whh'].T, gru[0]['bih'], gru[0]['bhh'],
          gru[1]['Wih'].T, gru[1]['Whh'].T, gru[1]['bih'], gru[1]['bhh'],
          params['temporal_output_norm'][0], params['temporal_output_norm'][1]]
    wp += _mlp_t(params['temporal_predictor'])
    fut = _temporal(lat, tip, ftp, wp)

    xf = fut.reshape(TOUT * N, D)
    xf = _run_blocks(xf, fc_blocks, pps[2:], knn_exp, R)

    v0 = velocity_in[0, -1]
    outs = _decoder(xf.reshape(TOUT, N, D), v0, _mlp_t(params['decoder']))
    return outs[None]


# speedup vs baseline: 11.3846x; 3.2431x over previous
import functools
import math

import jax
import jax.numpy as jnp
from jax import lax
from jax.experimental import pallas as pl
from jax.experimental.pallas import tpu as pltpu
from jax.experimental.pallas import tpu_sc as plsc

B = 1
N = 8192
K = 16
D = 128
MODES = 256
HEADS = 8
DH = D // HEADS
TIN = 5
TOUT = 5
NF = 16
TF = 2 * NF + 1
TILE = 256
NT = N // TILE


def _ln(x, g, b):
    m = x.mean(-1, keepdims=True)
    v = ((x - m) ** 2).mean(-1, keepdims=True)
    return (x - m) * lax.rsqrt(v + 1e-5) * g + b


def _gelu(x):
    c = math.sqrt(2.0 / math.pi)
    return 0.5 * x * (1.0 + jnp.tanh(c * (x + 0.044715 * (x ** 3))))


def _full(spec_shape):
    nd = len(spec_shape)
    return pl.BlockSpec(spec_shape, lambda i, _nd=nd: (0,) * _nd)


def _enc_body(f_ref, w1, b1, w2, b2, w3, b3, o_ref):
    x = _gelu(jnp.dot(f_ref[...], w1[...], preferred_element_type=jnp.float32) + b1[...])
    x = _gelu(jnp.dot(x, w2[...], preferred_element_type=jnp.float32) + b2[...])
    o_ref[...] = jnp.dot(x, w3[...], preferred_element_type=jnp.float32) + b3[...]


def _encoder(feats, wp):
    R = feats.shape[0]
    w1, b1, w2, b2, w3, b3 = wp
    return pl.pallas_call(
        _enc_body,
        grid=(R // TILE,),
        in_specs=[
            pl.BlockSpec((TILE, feats.shape[1]), lambda i: (i, 0)),
            _full(w1.shape), _full(b1.shape), _full(w2.shape),
            _full(b2.shape), _full(w3.shape), _full(b3.shape),
        ],
        out_specs=pl.BlockSpec((TILE, D), lambda i: (i, 0)),
        out_shape=jax.ShapeDtypeStruct((R, D), jnp.float32),
    )(feats, w1, b1, w2, b2, w3, b3)


def _stage_a_body(x_ref, pp_ref, g1, bb1, wh, h_ref, g_ref):
    h = _ln(x_ref[...], g1[...], bb1[...])
    h_ref[...] = h
    g_ref[...] = jnp.dot(h, wh[...], preferred_element_type=jnp.float32) + pp_ref[...]


def _stage_a(x, pp, bp):
    R = x.shape[0]
    return pl.pallas_call(
        _stage_a_body,
        grid=(R // TILE,),
        in_specs=[
            pl.BlockSpec((TILE, D), lambda i: (i, 0)),
            pl.BlockSpec((TILE, D), lambda i: (i % NT, 0)),
            _full((D,)), _full((D,)), _full((D, D)),
        ],
        out_specs=[
            pl.BlockSpec((TILE, D), lambda i: (i, 0)),
            pl.BlockSpec((TILE, D), lambda i: (i, 0)),
        ],
        out_shape=[
            jax.ShapeDtypeStruct((R, D), jnp.float32),
            jax.ShapeDtypeStruct((R, D), jnp.float32),
        ],
    )(x, pp, bp['ln1g'], bp['ln1b'], bp['wh_t'])


def _stage_b_body(gath_ref, x_ref, h_ref, pp_ref,
                  bmsg, w1, w2, bupd, g2, bb2, wqkbd, vbd, den, exp_m,
                  g3, bb3, wf1, bf1, wf2, bf2, o_ref):
    gath = gath_ref[...].reshape(TILE, K, D)
    c = pp_ref[...] - bmsg[...]
    msg = _gelu(gath - c[:, None, :])
    agg = jnp.mean(msg, axis=1)
    x1 = (x_ref[...]
          + jnp.dot(h_ref[...], w1[...], preferred_element_type=jnp.float32)
          + jnp.dot(agg, w2[...], preferred_element_type=jnp.float32)
          + bupd[...])
    h2 = _ln(x1, g2[...], bb2[...])
    s = jnp.dot(h2, wqkbd[...], preferred_element_type=jnp.float32)
    e = jnp.exp(s)
    denom = jnp.dot(e, den[...], preferred_element_type=jnp.float32)
    dfull = jnp.dot(denom, exp_m[...], preferred_element_type=jnp.float32)
    numer = jnp.dot(e, vbd[...], preferred_element_type=jnp.float32)
    x2 = x1 + numer / dfull
    h3 = _ln(x2, g3[...], bb3[...])
    f = _gelu(jnp.dot(h3, wf1[...], preferred_element_type=jnp.float32) + bf1[...])
    o_ref[...] = x2 + jnp.dot(f, wf2[...], preferred_element_type=jnp.float32) + bf2[...]


def _stage_b(gath, x, h, pp, bp):
    R = x.shape[0]
    return pl.pallas_call(
        _stage_b_body,
        grid=(R // TILE,),
        in_specs=[
            pl.BlockSpec((TILE * K, D), lambda i: (i, 0)),
            pl.BlockSpec((TILE, D), lambda i: (i, 0)),
            pl.BlockSpec((TILE, D), lambda i: (i, 0)),
            pl.BlockSpec((TILE, D), lambda i: (i % NT, 0)),
            _full((D,)), _full((D, D)), _full((D, D)), _full((D,)),
            _full((D,)), _full((D,)),
            _full((D, HEADS * MODES)), _full((HEADS * MODES, D)),
            _full((HEADS * MODES, HEADS)), _full((HEADS, D)),
            _full((D,)), _full((D,)),
            _full((D, 2 * D)), _full((2 * D,)), _full((2 * D, D)), _full((D,)),
        ],
        out_specs=pl.BlockSpec((TILE, D), lambda i: (i, 0)),
        out_shape=jax.ShapeDtypeStruct((R, D), jnp.float32),
    )(gath, x, h, pp,
      bp['bmsg'], bp['w1_t'], bp['w2_t'], bp['bupd'],
      bp['ln2g'], bp['ln2b'], bp['wqkbd'], bp['vbd'], bp['den'], bp['exp_m'],
      bp['ln3g'], bp['ln3b'], bp['wf1_t'], bp['bf1'], bp['wf2_t'], bp['bf2'])


_SC_CH = 256
_SC_NBUF = 2


def _sc_gather_body(g_hbm, idx_hbm, out_hbm, idx_v, rows_v, sems, nrows_w):
    wid = lax.axis_index("s") * 2 + lax.axis_index("c")
    base = wid * nrows_w
    nch = nrows_w // _SC_CH

    def start(ci, b):
        pltpu.sync_copy(idx_hbm.at[pl.ds(base + ci * _SC_CH, _SC_CH)], idx_v.at[b])
        pltpu.async_copy(g_hbm.at[idx_v.at[b]], rows_v.at[b], sems.at[b])

    def drain(ci, b):
        pltpu.make_async_copy(g_hbm.at[idx_v.at[b]], rows_v.at[b], sems.at[b]).wait()
        pltpu.sync_copy(rows_v.at[b], out_hbm.at[pl.ds(base + ci * _SC_CH, _SC_CH)])

    for b in range(_SC_NBUF):
        start(b, b)

    @pl.loop(0, nch, step=_SC_NBUF)
    def _(ci):
        for b in range(_SC_NBUF):
            drain(ci + b, b)

            @pl.when(ci + b + _SC_NBUF < nch)
            def _():
                start(ci + b + _SC_NBUF, b)


def _gather_rows(g, idx_flat):
    nw = 32
    nrows = idx_flat.shape[0]
    nrows_w = nrows // nw
    mesh = plsc.VectorSubcoreMesh(core_axis_name="c", subcore_axis_name="s")
    body = functools.partial(_sc_gather_body, nrows_w=nrows_w)
    return pl.kernel(
        body,
        out_type=jax.ShapeDtypeStruct((nrows, D), jnp.float32),
        mesh=mesh,
        scratch_types=[
            pltpu.VMEM((_SC_NBUF, _SC_CH), jnp.int32),
            pltpu.VMEM((_SC_NBUF, _SC_CH, D), jnp.float32),
            pltpu.SemaphoreType.DMA((_SC_NBUF,)),
        ],
        compiler_params=pltpu.CompilerParams(use_tc_tiling_on_sc=False),
    )(g, idx_flat)


def _gru_body(lat_ref, tip_ref, gin, bin_, w_ih0, w_hh0, bih0, bhh0,
              w_ih1, w_hh1, bih1, bhh1, gout, bout,
              pw1, pb1, pw2, pb2, pw3, pb3, ftp_ref, o_ref):
    xs = [_ln(lat_ref[t] + tip_ref[t], gin[...], bin_[...]) for t in range(TIN)]
    for (wi, wh, bi, bh) in ((w_ih0, w_hh0, bih0, bhh0), (w_ih1, w_hh1, bih1, bhh1)):
        h = jnp.zeros((TILE, D), jnp.float32)
        ys = []
        for t in range(TIN):
            gi = jnp.dot(xs[t], wi[...], preferred_element_type=jnp.float32) + bi[...]
            gh = jnp.dot(h, wh[...], preferred_element_type=jnp.float32) + bh[...]
            r = jax.nn.sigmoid(gi[:, :D] + gh[:, :D])
            z = jax.nn.sigmoid(gi[:, D:2 * D] + gh[:, D:2 * D])
            nn = jnp.tanh(gi[:, 2 * D:] + r * gh[:, 2 * D:])
            h = (1.0 - z) * nn + z * h
            ys.append(h)
        xs = ys
    seq = [_ln(y, gout[...], bout[...]) for y in xs]
    flat = jnp.concatenate(seq, axis=-1)
    p = _gelu(jnp.dot(flat, pw1[...], preferred_element_type=jnp.float32) + pb1[...])
    p = _gelu(jnp.dot(p, pw2[...], preferred_element_type=jnp.float32) + pb2[...])
    p = jnp.dot(p, pw3[...], preferred_element_type=jnp.float32) + pb3[...]
    for s in range(TOUT):
        o_ref[s] = p[:, s * D:(s + 1) * D] + ftp_ref[s]


def _temporal(lat, tip, ftp, wp):
    args = [lat, tip] + list(wp) + [ftp]
    in_specs = [pl.BlockSpec((TIN, TILE, D), lambda i: (0, i, 0)),
                _full((TIN, D))]
    in_specs += [_full(w.shape) for w in wp]
    in_specs += [_full((TOUT, D))]
    return pl.pallas_call(
        _gru_body,
        grid=(NT,),
        in_specs=in_specs,
        out_specs=pl.BlockSpec((TOUT, TILE, D), lambda i: (0, i, 0)),
        out_shape=jax.ShapeDtypeStruct((TOUT, N, D), jnp.float32),
    )(*args)


def _dec_body(x_ref, v0_ref, w1, b1, w2, b2, w3, b3, o_ref):
    acc = v0_ref[...]
    for s in range(TOUT):
        y = _gelu(jnp.dot(x_ref[s], w1[...], preferred_element_type=jnp.float32) + b1[...])
        y = _gelu(jnp.dot(y, w2[...], preferred_element_type=jnp.float32) + b2[...])
        y = jnp.dot(y, w3[...], preferred_element_type=jnp.float32) + b3[...]
        acc = acc + y
        o_ref[s] = acc


def _decoder(x, v0, wp):
    w1, b1, w2, b2, w3, b3 = wp
    return pl.pallas_call(
        _dec_body,
        grid=(NT,),
        in_specs=[
            pl.BlockSpec((TOUT, TILE, D), lambda i: (0, i, 0)),
            pl.BlockSpec((TILE, 3), lambda i: (i, 0)),
            _full(w1.shape), _full(b1.shape), _full(w2.shape),
            _full(b2.shape), _full(w3.shape), _full(b3.shape),
        ],
        out_specs=pl.BlockSpec((TOUT, TILE, 3), lambda i: (0, i, 0)),
        out_shape=jax.ShapeDtypeStruct((TOUT, N, 3), jnp.float32),
    )(x, v0, w1, b1, w2, b2, w3, b3)


def _pp_body(pos_ref, wr_ref, o_ref):
    o_ref[...] = jnp.dot(pos_ref[...], wr_ref[...], preferred_element_type=jnp.float32)


def _pos_proj(pos, wr_all):
    nb = wr_all.shape[1]
    return pl.pallas_call(
        _pp_body,
        grid=(NT,),
        in_specs=[pl.BlockSpec((TILE, 3), lambda i: (i, 0)), _full((3, nb))],
        out_specs=pl.BlockSpec((TILE, nb), lambda i: (i, 0)),
        out_shape=jax.ShapeDtypeStruct((N, nb), jnp.float32),
    )(pos, wr_all)


def _prep_block(p):
    wmsg = p['Wmsg']
    basis = p['basis']
    kb = basis.reshape(MODES, HEADS, DH)
    kbd = jax.scipy.linalg.block_diag(*[kb[:, h, :].T for h in range(HEADS)])
    vb = (basis @ p['Wv'].T).reshape(MODES, HEADS, DH)
    vbd = jax.scipy.linalg.block_diag(*[vb[:, h, :] for h in range(HEADS)])
    den = jnp.kron(jnp.eye(HEADS, dtype=jnp.float32), jnp.ones((MODES, 1), jnp.float32))
    exp_m = jnp.kron(jnp.eye(HEADS, dtype=jnp.float32), jnp.ones((1, DH), jnp.float32))
    (f1w, f1b), (f2w, f2b) = p['ffn']
    return {
        'ln1g': p['ln1'][0], 'ln1b': p['ln1'][1],
        'ln2g': p['ln2'][0], 'ln2b': p['ln2'][1],
        'ln3g': p['ln3'][0], 'ln3b': p['ln3'][1],
        'wh_t': wmsg[:, :D].T, 'wr_t': wmsg[:, D:].T, 'bmsg': p['bmsg'],
        'w1_t': p['Wupd'][:, :D].T, 'w2_t': p['Wupd'][:, D:].T, 'bupd': p['bupd'],
        'wqkbd': (p['Wq'].T @ kbd) / math.sqrt(DH), 'vbd': vbd,
        'den': den, 'exp_m': exp_m,
        'wf1_t': f1w.T, 'bf1': f1b, 'wf2_t': f2w.T, 'bf2': f2b,
    }


def _mlp_t(params):
    out = []
    for (w, b) in params:
        out.append(w.T)
        out.append(b)
    return out


def _mlp_jax(params, x):
    n = len(params)
    for i, (w, b) in enumerate(params):
        x = x @ w.T + b
        if i < n - 1:
            x = jax.nn.gelu(x)
    return x


def _fourier(t):
    freqs = jnp.pi * (2.0 ** jnp.arange(NF))
    a = t[..., None] * freqs
    return jnp.concatenate([t[..., None], jnp.sin(a), jnp.cos(a)], -1)


def _run_blocks(x, blocks, pps, knn_flat, R):
    for bp, pp in zip(blocks, pps):
        h, g = _stage_a(x, pp, bp)
        gath = _gather_rows(g, knn_flat)
        x = _stage_b(gath, x, h, pp, bp)
    return x


def kernel(t, pos, idcs_airfoil, velocity_in, wall_distance, surface_frame,
           knn_indices, params):
    temb = _fourier(t)
    in_emb = temb[:, :TIN]
    out_emb = temb[:, TIN:]
    tip = _mlp_jax(params['temporal_input_proj'], in_emb)[0]
    ftp = _mlp_jax(params['future_time_proj'], out_emb)[0]

    pos2 = pos[0]
    mask = jnp.zeros((N,), jnp.float32).at[idcs_airfoil[0]].set(1.0)
    wall = jnp.log1p(wall_distance[0])[:, None]
    sf = surface_frame[0]

    tfb = jnp.broadcast_to(in_emb[0][:, None, :], (TIN, N, TF))
    rest = jnp.concatenate([wall, mask[:, None], sf], -1)
    feats = jnp.concatenate([
        jnp.broadcast_to(pos2[None], (TIN, N, 3)),
        velocity_in[0],
        tfb,
        jnp.broadcast_to(rest[None], (TIN, N, 11)),
    ], -1).reshape(TIN * N, 3 + 3 + TF + 11)

    enc_blocks = [_prep_block(p) for p in params['encoder_blocks']]
    fc_blocks = [_prep_block(p) for p in params['forecast_blocks']]

    wr_all = jnp.concatenate([bp['wr_t'] for bp in enc_blocks + fc_blocks], axis=1)
    pp_all = _pos_proj(pos2, wr_all)
    pps = [pp_all[:, i * D:(i + 1) * D] for i in range(4)]

    knn = knn_indices[0].astype(jnp.int32)
    base = jnp.arange(TIN, dtype=jnp.int32)[:, None, None] * N
    knn_exp = (knn[None] + base).reshape(TIN * N * K)

    R = TIN * N

    x = _encoder(feats, _mlp_t(params['frame_encoder']))
    x = _run_blocks(x, enc_blocks, pps[:2], knn_exp, R)

    lat = x.reshape(TIN, N, D)
    gru = params['gru']
    wp = [params['temporal_input_norm'][0], params['temporal_input_norm'][1],
          gru[0]['Wih'].T, gru[0]['Whh'].T, gru[0]['bih'], gru[0]['bhh'],
          gru[1]['Wih'].T, gru[1]['Whh'].T, gru[1]['bih'], gru[1]['bhh'],
          params['temporal_output_norm'][0], params['temporal_output_norm'][1]]
    wp += _mlp_t(params['temporal_predictor'])
    fut = _temporal(lat, tip, ftp, wp)

    xf = fut.reshape(TOUT * N, D)
    xf = _run_blocks(xf, fc_blocks, pps[2:], knn_exp, R)

    v0 = velocity_in[0, -1]
    outs = _decoder(xf.reshape(TOUT, N, D), v0, _mlp_t(params['decoder']))
    return outs[None]


# speedup vs baseline: 13.9680x; 1.2269x over previous
import functools
import math

import jax
import jax.numpy as jnp
from jax import lax
from jax.experimental import pallas as pl
from jax.experimental.pallas import tpu as pltpu
from jax.experimental.pallas import tpu_sc as plsc

B = 1
N = 8192
K = 16
D = 128
MODES = 256
HEADS = 8
DH = D // HEADS
TIN = 5
TOUT = 5
NF = 16
TF = 2 * NF + 1
TILE = 256
NT = N // TILE


def _ln(x, g, b):
    m = x.mean(-1, keepdims=True)
    v = ((x - m) ** 2).mean(-1, keepdims=True)
    return (x - m) * lax.rsqrt(v + 1e-5) * g + b


def _gelu(x):
    c = math.sqrt(2.0 / math.pi)
    return 0.5 * x * (1.0 + jnp.tanh(c * (x + 0.044715 * (x ** 3))))


def _full(spec_shape):
    nd = len(spec_shape)
    return pl.BlockSpec(spec_shape, lambda i, _nd=nd: (0,) * _nd)


def _enc_body(f_ref, w1, b1, w2, b2, w3, b3, o_ref):
    x = _gelu(jnp.dot(f_ref[...], w1[...], preferred_element_type=jnp.float32) + b1[...])
    x = _gelu(jnp.dot(x, w2[...], preferred_element_type=jnp.float32) + b2[...])
    o_ref[...] = jnp.dot(x, w3[...], preferred_element_type=jnp.float32) + b3[...]


def _encoder(feats, wp):
    w1, b1, w2, b2, w3, b3 = wp
    return pl.pallas_call(
        _enc_body,
        grid=(NT,),
        in_specs=[
            pl.BlockSpec((TILE, feats.shape[1]), lambda i: (i, 0)),
            _full(w1.shape), _full(b1.shape), _full(w2.shape),
            _full(b2.shape), _full(w3.shape), _full(b3.shape),
        ],
        out_specs=pl.BlockSpec((TILE, D), lambda i: (i, 0)),
        out_shape=jax.ShapeDtypeStruct((N, D), jnp.float32),
    )(feats, w1, b1, w2, b2, w3, b3)


def _stage_a_body(x_ref, pp_ref, g1, bb1, wh, h_ref, g_ref):
    h = _ln(x_ref[...], g1[...], bb1[...])
    h_ref[...] = h
    g_ref[...] = jnp.dot(h, wh[...], preferred_element_type=jnp.float32) + pp_ref[...]


def _stage_a(x, pp, bp):
    return pl.pallas_call(
        _stage_a_body,
        grid=(NT,),
        in_specs=[
            pl.BlockSpec((TILE, D), lambda i: (i, 0)),
            pl.BlockSpec((TILE, D), lambda i: (i, 0)),
            _full((D,)), _full((D,)), _full((D, D)),
        ],
        out_specs=[
            pl.BlockSpec((TILE, D), lambda i: (i, 0)),
            pl.BlockSpec((TILE, D), lambda i: (i, 0)),
        ],
        out_shape=[
            jax.ShapeDtypeStruct((N, D), jnp.float32),
            jax.ShapeDtypeStruct((N, D), jnp.float32),
        ],
    )(x, pp, bp['ln1g'], bp['ln1b'], bp['wh_t'])


def _stage_b_body(gath_ref, x_ref, h_ref, pp_ref,
                  bmsg, w1, w2, bupd, g2, bb2, wqkbd, vbd, den, exp_m,
                  g3, bb3, wf1, bf1, wf2, bf2, o_ref):
    gath = gath_ref[...].reshape(TILE, K, D)
    c = pp_ref[...] - bmsg[...]
    msg = _gelu(gath - c[:, None, :])
    agg = jnp.mean(msg, axis=1)
    x1 = (x_ref[...]
          + jnp.dot(h_ref[...], w1[...], preferred_element_type=jnp.float32)
          + jnp.dot(agg, w2[...], preferred_element_type=jnp.float32)
          + bupd[...])
    h2 = _ln(x1, g2[...], bb2[...])
    s = jnp.dot(h2.astype(jnp.bfloat16), wqkbd[...],
                preferred_element_type=jnp.float32)
    e = jnp.exp(s)
    eb = e.astype(jnp.bfloat16)
    denom = jnp.dot(eb, den[...], preferred_element_type=jnp.float32)
    dfull = jnp.dot(denom, exp_m[...], preferred_element_type=jnp.float32)
    numer = jnp.dot(eb, vbd[...], preferred_element_type=jnp.float32)
    x2 = x1 + numer / dfull
    h3 = _ln(x2, g3[...], bb3[...])
    f = _gelu(jnp.dot(h3, wf1[...], preferred_element_type=jnp.float32) + bf1[...])
    o_ref[...] = x2 + jnp.dot(f, wf2[...], preferred_element_type=jnp.float32) + bf2[...]


def _stage_b(gath, x, h, pp, bp):
    return pl.pallas_call(
        _stage_b_body,
        grid=(NT,),
        in_specs=[
            pl.BlockSpec((TILE * K, D), lambda i: (i, 0)),
            pl.BlockSpec((TILE, D), lambda i: (i, 0)),
            pl.BlockSpec((TILE, D), lambda i: (i, 0)),
            pl.BlockSpec((TILE, D), lambda i: (i, 0)),
            _full((D,)), _full((D, D)), _full((D, D)), _full((D,)),
            _full((D,)), _full((D,)),
            _full((D, HEADS * MODES)), _full((HEADS * MODES, D)),
            _full((HEADS * MODES, HEADS)), _full((HEADS, D)),
            _full((D,)), _full((D,)),
            _full((D, 2 * D)), _full((2 * D,)), _full((2 * D, D)), _full((D,)),
        ],
        out_specs=pl.BlockSpec((TILE, D), lambda i: (i, 0)),
        out_shape=jax.ShapeDtypeStruct((N, D), jnp.float32),
    )(gath, x, h, pp,
      bp['bmsg'], bp['w1_t'], bp['w2_t'], bp['bupd'],
      bp['ln2g'], bp['ln2b'], bp['wqkbd'], bp['vbd'], bp['den'], bp['exp_m'],
      bp['ln3g'], bp['ln3b'], bp['wf1_t'], bp['bf1'], bp['wf2_t'], bp['bf2'])


_SC_CH = 128
_SC_NBUF = 4


def _sc_gather_body(g_hbm, idx_hbm, out_hbm, idx_v, rows_v, sems_g, sems_o,
                    nrows_w):
    wid = lax.axis_index("s") * 2 + lax.axis_index("c")
    base = wid * nrows_w
    nch = nrows_w // _SC_CH
    pltpu.sync_copy(idx_hbm.at[pl.ds(base, nrows_w)], idx_v)

    def gstart(i, b):
        pltpu.async_copy(g_hbm.at[idx_v.at[pl.ds(i * _SC_CH, _SC_CH)]],
                         rows_v.at[b], sems_g.at[b])

    def gwait(i, b):
        pltpu.make_async_copy(g_hbm.at[idx_v.at[pl.ds(i * _SC_CH, _SC_CH)]],
                              rows_v.at[b], sems_g.at[b]).wait()

    def ostart(i, b):
        pltpu.async_copy(rows_v.at[b],
                         out_hbm.at[pl.ds(base + i * _SC_CH, _SC_CH)],
                         sems_o.at[b])

    def owait(i, b):
        pltpu.make_async_copy(rows_v.at[b],
                              out_hbm.at[pl.ds(base + i * _SC_CH, _SC_CH)],
                              sems_o.at[b]).wait()

    for b in range(_SC_NBUF):
        gstart(b, b)

    @pl.loop(0, nch, step=_SC_NBUF)
    def _(ci):
        for b in range(_SC_NBUF):
            gwait(ci + b, b)
            ostart(ci + b, b)
        for b in range(_SC_NBUF):
            @pl.when(ci + b + _SC_NBUF < nch)
            def _():
                owait(ci + b, b)
                gstart(ci + b + _SC_NBUF, b)

    for b in range(_SC_NBUF):
        owait(nch - _SC_NBUF + b, b)


def _gather_rows(g, idx_flat):
    nw = 32
    nrows = idx_flat.shape[0]
    nrows_w = nrows // nw
    mesh = plsc.VectorSubcoreMesh(core_axis_name="c", subcore_axis_name="s")
    body = functools.partial(_sc_gather_body, nrows_w=nrows_w)
    return pl.kernel(
        body,
        out_type=jax.ShapeDtypeStruct((nrows, D), jnp.float32),
        mesh=mesh,
        scratch_types=[
            pltpu.VMEM((nrows_w,), jnp.int32),
            pltpu.VMEM((_SC_NBUF, _SC_CH, D), jnp.float32),
            pltpu.SemaphoreType.DMA((_SC_NBUF,)),
            pltpu.SemaphoreType.DMA((_SC_NBUF,)),
        ],
        compiler_params=pltpu.CompilerParams(use_tc_tiling_on_sc=False),
    )(g, idx_flat)


def _gru_body(l0, l1, l2, l3, l4, tip_ref, gin, bin_, w_ih0, w_hh0, bih0, bhh0,
              w_ih1, w_hh1, bih1, bhh1, gout, bout,
              pw1, pb1, pw2, pb2, pw3, pb3, ftp_ref,
              o0, o1, o2, o3, o4):
    lrefs = (l0, l1, l2, l3, l4)
    xs = [_ln(lrefs[t][...] + tip_ref[t], gin[...], bin_[...]) for t in range(TIN)]
    for (wi, wh, bi, bh) in ((w_ih0, w_hh0, bih0, bhh0), (w_ih1, w_hh1, bih1, bhh1)):
        h = jnp.zeros((TILE, D), jnp.float32)
        ys = []
        for t in range(TIN):
            gi = jnp.dot(xs[t], wi[...], preferred_element_type=jnp.float32) + bi[...]
            gh = jnp.dot(h, wh[...], preferred_element_type=jnp.float32) + bh[...]
            r = jax.nn.sigmoid(gi[:, :D] + gh[:, :D])
            z = jax.nn.sigmoid(gi[:, D:2 * D] + gh[:, D:2 * D])
            nn = jnp.tanh(gi[:, 2 * D:] + r * gh[:, 2 * D:])
            h = (1.0 - z) * nn + z * h
            ys.append(h)
        xs = ys
    seq = [_ln(y, gout[...], bout[...]) for y in xs]
    flat = jnp.concatenate(seq, axis=-1)
    p = _gelu(jnp.dot(flat, pw1[...], preferred_element_type=jnp.float32) + pb1[...])
    p = _gelu(jnp.dot(p, pw2[...], preferred_element_type=jnp.float32) + pb2[...])
    p = jnp.dot(p, pw3[...], preferred_element_type=jnp.float32) + pb3[...]
    orefs = (o0, o1, o2, o3, o4)
    for s in range(TOUT):
        orefs[s][...] = p[:, s * D:(s + 1) * D] + ftp_ref[s]


def _temporal(lats, tip, ftp, wp):
    args = list(lats) + [tip] + list(wp) + [ftp]
    row_spec = pl.BlockSpec((TILE, D), lambda i: (i, 0))
    in_specs = [row_spec] * TIN + [_full((TIN, D))]
    in_specs += [_full(w.shape) for w in wp]
    in_specs += [_full((TOUT, D))]
    return pl.pallas_call(
        _gru_body,
        grid=(NT,),
        in_specs=in_specs,
        out_specs=[row_spec] * TOUT,
        out_shape=[jax.ShapeDtypeStruct((N, D), jnp.float32)] * TOUT,
    )(*args)


def _dec_body(x0, x1, x2, x3, x4, v0_ref, w1, b1, w2, b2, w3, b3, o_ref):
    acc = v0_ref[...]
    xrefs = (x0, x1, x2, x3, x4)
    for s in range(TOUT):
        y = _gelu(jnp.dot(xrefs[s][...], w1[...], preferred_element_type=jnp.float32) + b1[...])
        y = _gelu(jnp.dot(y, w2[...], preferred_element_type=jnp.float32) + b2[...])
        y = jnp.dot(y, w3[...], preferred_element_type=jnp.float32) + b3[...]
        acc = acc + y
        o_ref[s] = acc


def _decoder(xs, v0, wp):
    w1, b1, w2, b2, w3, b3 = wp
    row_spec = pl.BlockSpec((TILE, D), lambda i: (i, 0))
    return pl.pallas_call(
        _dec_body,
        grid=(NT,),
        in_specs=[row_spec] * TOUT + [
            pl.BlockSpec((TILE, 3), lambda i: (i, 0)),
            _full(w1.shape), _full(b1.shape), _full(w2.shape),
            _full(b2.shape), _full(w3.shape), _full(b3.shape),
        ],
        out_specs=pl.BlockSpec((TOUT, TILE, 3), lambda i: (0, i, 0)),
        out_shape=jax.ShapeDtypeStruct((TOUT, N, 3), jnp.float32),
    )(*xs, v0, w1, b1, w2, b2, w3, b3)


def _pp_body(pos_ref, wr_ref, o_ref):
    o_ref[...] = jnp.dot(pos_ref[...], wr_ref[...], preferred_element_type=jnp.float32)


def _pos_proj(pos, wr_all):
    nb = wr_all.shape[1]
    return pl.pallas_call(
        _pp_body,
        grid=(NT,),
        in_specs=[pl.BlockSpec((TILE, 3), lambda i: (i, 0)), _full((3, nb))],
        out_specs=pl.BlockSpec((TILE, nb), lambda i: (i, 0)),
        out_shape=jax.ShapeDtypeStruct((N, nb), jnp.float32),
    )(pos, wr_all)


def _prep_block(p):
    wmsg = p['Wmsg']
    basis = p['basis']
    kb = basis.reshape(MODES, HEADS, DH)
    kbd = jax.scipy.linalg.block_diag(*[kb[:, h, :].T for h in range(HEADS)])
    vb = (basis @ p['Wv'].T).reshape(MODES, HEADS, DH)
    vbd = jax.scipy.linalg.block_diag(*[vb[:, h, :] for h in range(HEADS)])
    den = jnp.kron(jnp.eye(HEADS, dtype=jnp.float32), jnp.ones((MODES, 1), jnp.float32))
    exp_m = jnp.kron(jnp.eye(HEADS, dtype=jnp.float32), jnp.ones((1, DH), jnp.float32))
    (f1w, f1b), (f2w, f2b) = p['ffn']
    return {
        'ln1g': p['ln1'][0], 'ln1b': p['ln1'][1],
        'ln2g': p['ln2'][0], 'ln2b': p['ln2'][1],
        'ln3g': p['ln3'][0], 'ln3b': p['ln3'][1],
        'wh_t': wmsg[:, :D].T, 'wr_t': wmsg[:, D:].T, 'bmsg': p['bmsg'],
        'w1_t': p['Wupd'][:, :D].T, 'w2_t': p['Wupd'][:, D:].T, 'bupd': p['bupd'],
        'wqkbd': ((p['Wq'].T @ kbd) / math.sqrt(DH)).astype(jnp.bfloat16),
        'vbd': vbd.astype(jnp.bfloat16),
        'den': den.astype(jnp.bfloat16), 'exp_m': exp_m,
        'wf1_t': f1w.T, 'bf1': f1b, 'wf2_t': f2w.T, 'bf2': f2b,
    }


def _mlp_t(params):
    out = []
    for (w, b) in params:
        out.append(w.T)
        out.append(b)
    return out


def _mlp_jax(params, x):
    n = len(params)
    for i, (w, b) in enumerate(params):
        x = x @ w.T + b
        if i < n - 1:
            x = jax.nn.gelu(x)
    return x


def _fourier(t):
    freqs = jnp.pi * (2.0 ** jnp.arange(NF))
    a = t[..., None] * freqs
    return jnp.concatenate([t[..., None], jnp.sin(a), jnp.cos(a)], -1)


def _run_chain(x, blocks, pps, knn_flat):
    for bp, pp in zip(blocks, pps):
        h, g = _stage_a(x, pp, bp)
        gath = _gather_rows(g, knn_flat)
        x = _stage_b(gath, x, h, pp, bp)
    return x


def kernel(t, pos, idcs_airfoil, velocity_in, wall_distance, surface_frame,
           knn_indices, params):
    temb = _fourier(t)
    in_emb = temb[:, :TIN]
    out_emb = temb[:, TIN:]
    tip = _mlp_jax(params['temporal_input_proj'], in_emb)[0]
    ftp = _mlp_jax(params['future_time_proj'], out_emb)[0]

    pos2 = pos[0]
    mask = jnp.zeros((N,), jnp.float32).at[idcs_airfoil[0]].set(1.0)
    wall = jnp.log1p(wall_distance[0])[:, None]
    sf = surface_frame[0]

    rest = jnp.concatenate([wall, mask[:, None], sf], -1)
    feats = [jnp.concatenate([
        pos2,
        velocity_in[0, s],
        jnp.broadcast_to(in_emb[0, s][None, :], (N, TF)),
        rest,
    ], -1) for s in range(TIN)]

    enc_blocks = [_prep_block(p) for p in params['encoder_blocks']]
    fc_blocks = [_prep_block(p) for p in params['forecast_blocks']]

    wr_all = jnp.concatenate([bp['wr_t'] for bp in enc_blocks + fc_blocks], axis=1)
    pp_all = _pos_proj(pos2, wr_all)
    pps = [pp_all[:, i * D:(i + 1) * D] for i in range(4)]

    knn_flat = knn_indices[0].astype(jnp.int32).reshape(N * K)

    enc_w = _mlp_t(params['frame_encoder'])
    lats = [_run_chain(_encoder(feats[s], enc_w), enc_blocks, pps[:2], knn_flat)
            for s in range(TIN)]

    gru = params['gru']
    wp = [params['temporal_input_norm'][0], params['temporal_input_norm'][1],
          gru[0]['Wih'].T, gru[0]['Whh'].T, gru[0]['bih'], gru[0]['bhh'],
          gru[1]['Wih'].T, gru[1]['Whh'].T, gru[1]['bih'], gru[1]['bhh'],
          params['temporal_output_norm'][0], params['temporal_output_norm'][1]]
    wp += _mlp_t(params['temporal_predictor'])
    futs = _temporal(lats, tip, ftp, wp)

    xfs = [_run_chain(f, fc_blocks, pps[2:], knn_flat) for f in futs]

    v0 = velocity_in[0, -1]
    outs = _decoder(xfs, v0, _mlp_t(params['decoder']))
    return outs[None]


# speedup vs baseline: 15.5601x; 1.1140x over previous
import functools
import math

import jax
import jax.numpy as jnp
from jax import lax
from jax.experimental import pallas as pl
from jax.experimental.pallas import tpu as pltpu
from jax.experimental.pallas import tpu_sc as plsc

B = 1
N = 8192
K = 16
D = 128
MODES = 256
HEADS = 8
DH = D // HEADS
TIN = 5
TOUT = 5
NF = 16
TF = 2 * NF + 1
TILE = 256
NT = N // TILE


def _ln(x, g, b):
    m = x.mean(-1, keepdims=True)
    v = ((x - m) ** 2).mean(-1, keepdims=True)
    return (x - m) * lax.rsqrt(v + 1e-5) * g + b


def _gelu(x):
    c = math.sqrt(2.0 / math.pi)
    return 0.5 * x * (1.0 + jnp.tanh(c * (x + 0.044715 * (x ** 3))))


def _full(spec_shape):
    nd = len(spec_shape)
    return pl.BlockSpec(spec_shape, lambda i, _nd=nd: (0,) * _nd)


def _enc_body(f_ref, w1, b1, w2, b2, w3, b3, pp_ref, g1, bb1, wh,
              o_ref, h_ref, g_ref):
    x = _gelu(jnp.dot(f_ref[...], w1[...], preferred_element_type=jnp.float32) + b1[...])
    x = _gelu(jnp.dot(x, w2[...], preferred_element_type=jnp.float32) + b2[...])
    x = jnp.dot(x, w3[...], preferred_element_type=jnp.float32) + b3[...]
    o_ref[...] = x
    h = _ln(x, g1[...], bb1[...])
    h_ref[...] = h
    g_ref[...] = jnp.dot(h, wh[...], preferred_element_type=jnp.float32) + pp_ref[...]


def _encoder(feats, wp, pp, bp):
    w1, b1, w2, b2, w3, b3 = wp
    row_spec = pl.BlockSpec((TILE, D), lambda i: (i, 0))
    return pl.pallas_call(
        _enc_body,
        grid=(NT,),
        in_specs=[
            pl.BlockSpec((TILE, feats.shape[1]), lambda i: (i, 0)),
            _full(w1.shape), _full(b1.shape), _full(w2.shape),
            _full(b2.shape), _full(w3.shape), _full(b3.shape),
            row_spec, _full((D,)), _full((D,)), _full((D, D)),
        ],
        out_specs=[row_spec] * 3,
        out_shape=[jax.ShapeDtypeStruct((N, D), jnp.float32)] * 3,
    )(feats, w1, b1, w2, b2, w3, b3, pp, bp['ln1g'], bp['ln1b'], bp['wh_t'])


def _stage_a_body(x_ref, pp_ref, g1, bb1, wh, h_ref, g_ref):
    h = _ln(x_ref[...], g1[...], bb1[...])
    h_ref[...] = h
    g_ref[...] = jnp.dot(h, wh[...], preferred_element_type=jnp.float32) + pp_ref[...]


def _stage_a(x, pp, bp):
    return pl.pallas_call(
        _stage_a_body,
        grid=(NT,),
        in_specs=[
            pl.BlockSpec((TILE, D), lambda i: (i, 0)),
            pl.BlockSpec((TILE, D), lambda i: (i, 0)),
            _full((D,)), _full((D,)), _full((D, D)),
        ],
        out_specs=[
            pl.BlockSpec((TILE, D), lambda i: (i, 0)),
            pl.BlockSpec((TILE, D), lambda i: (i, 0)),
        ],
        out_shape=[
            jax.ShapeDtypeStruct((N, D), jnp.float32),
            jax.ShapeDtypeStruct((N, D), jnp.float32),
        ],
    )(x, pp, bp['ln1g'], bp['ln1b'], bp['wh_t'])


def _stage_b_body(gath_ref, x_ref, h_ref, pp_ref,
                  bmsg, w1, w2, bupd, g2, bb2, wqkbd, vbd, den, exp_m,
                  g3, bb3, wf1, bf1, wf2, bf2, *rest):
    gath = gath_ref[...].reshape(TILE, K, D)
    c = pp_ref[...] - bmsg[...]
    msg = _gelu(gath - c[:, None, :])
    agg = jnp.mean(msg, axis=1)
    x1 = (x_ref[...]
          + jnp.dot(h_ref[...], w1[...], preferred_element_type=jnp.float32)
          + jnp.dot(agg, w2[...], preferred_element_type=jnp.float32)
          + bupd[...])
    h2 = _ln(x1, g2[...], bb2[...])
    s = jnp.dot(h2.astype(jnp.bfloat16), wqkbd[...],
                preferred_element_type=jnp.float32)
    e = jnp.exp(s)
    eb = e.astype(jnp.bfloat16)
    denom = jnp.dot(eb, den[...], preferred_element_type=jnp.float32)
    dfull = jnp.dot(denom, exp_m[...], preferred_element_type=jnp.float32)
    numer = jnp.dot(eb, vbd[...], preferred_element_type=jnp.float32)
    x2 = x1 + numer / dfull
    h3 = _ln(x2, g3[...], bb3[...])
    f = _gelu(jnp.dot(h3, wf1[...], preferred_element_type=jnp.float32) + bf1[...])
    x3 = x2 + jnp.dot(f, wf2[...], preferred_element_type=jnp.float32) + bf2[...]
    if len(rest) == 1:
        rest[0][...] = x3
    else:
        ppn, g1n, bb1n, whn, o_ref, hn_ref, gn_ref = rest
        o_ref[...] = x3
        hn = _ln(x3, g1n[...], bb1n[...])
        hn_ref[...] = hn
        gn_ref[...] = (jnp.dot(hn, whn[...], preferred_element_type=jnp.float32)
                       + ppn[...])


def _stage_b(gath, x, h, pp, bp, tail=None):
    row_spec = pl.BlockSpec((TILE, D), lambda i: (i, 0))
    in_specs = [
        pl.BlockSpec((TILE * K, D), lambda i: (i, 0)),
        row_spec, row_spec, row_spec,
        _full((D,)), _full((D, D)), _full((D, D)), _full((D,)),
        _full((D,)), _full((D,)),
        _full((D, HEADS * MODES)), _full((HEADS * MODES, D)),
        _full((HEADS * MODES, HEADS)), _full((HEADS, D)),
        _full((D,)), _full((D,)),
        _full((D, 2 * D)), _full((2 * D,)), _full((2 * D, D)), _full((D,)),
    ]
    args = [gath, x, h, pp,
            bp['bmsg'], bp['w1_t'], bp['w2_t'], bp['bupd'],
            bp['ln2g'], bp['ln2b'], bp['wqkbd'], bp['vbd'], bp['den'],
            bp['exp_m'],
            bp['ln3g'], bp['ln3b'], bp['wf1_t'], bp['bf1'], bp['wf2_t'],
            bp['bf2']]
    if tail is None:
        n_out = 1
    else:
        ppn, bpn = tail
        in_specs += [row_spec, _full((D,)), _full((D,)), _full((D, D))]
        args += [ppn, bpn['ln1g'], bpn['ln1b'], bpn['wh_t']]
        n_out = 3
    return pl.pallas_call(
        _stage_b_body,
        grid=(NT,),
        in_specs=in_specs,
        out_specs=[row_spec] * n_out,
        out_shape=[jax.ShapeDtypeStruct((N, D), jnp.float32)] * n_out,
    )(*args)


_SC_CH = 128
_SC_NBUF = 4


def _sc_gather_body(g_hbm, idx_hbm, out_hbm, idx_v, rows_v, sems_g, sems_o,
                    nrows_w):
    wid = lax.axis_index("s") * 2 + lax.axis_index("c")
    base = wid * nrows_w
    nch = nrows_w // _SC_CH
    pltpu.sync_copy(idx_hbm.at[pl.ds(base, nrows_w)], idx_v)

    def gstart(i, b):
        pltpu.async_copy(g_hbm.at[idx_v.at[pl.ds(i * _SC_CH, _SC_CH)]],
                         rows_v.at[b], sems_g.at[b])

    def gwait(i, b):
        pltpu.make_async_copy(g_hbm.at[idx_v.at[pl.ds(i * _SC_CH, _SC_CH)]],
                              rows_v.at[b], sems_g.at[b]).wait()

    def ostart(i, b):
        pltpu.async_copy(rows_v.at[b],
                         out_hbm.at[pl.ds(base + i * _SC_CH, _SC_CH)],
                         sems_o.at[b])

    def owait(i, b):
        pltpu.make_async_copy(rows_v.at[b],
                              out_hbm.at[pl.ds(base + i * _SC_CH, _SC_CH)],
                              sems_o.at[b]).wait()

    for b in range(_SC_NBUF):
        gstart(b, b)

    @pl.loop(0, nch, step=_SC_NBUF)
    def _(ci):
        for b in range(_SC_NBUF):
            gwait(ci + b, b)
            ostart(ci + b, b)
        for b in range(_SC_NBUF):
            @pl.when(ci + b + _SC_NBUF < nch)
            def _():
                owait(ci + b, b)
                gstart(ci + b + _SC_NBUF, b)

    for b in range(_SC_NBUF):
        owait(nch - _SC_NBUF + b, b)


def _gather_rows(g, idx_flat):
    nw = 32
    nrows = idx_flat.shape[0]
    nrows_w = nrows // nw
    mesh = plsc.VectorSubcoreMesh(core_axis_name="c", subcore_axis_name="s")
    body = functools.partial(_sc_gather_body, nrows_w=nrows_w)
    return pl.kernel(
        body,
        out_type=jax.ShapeDtypeStruct((nrows, D), jnp.float32),
        mesh=mesh,
        scratch_types=[
            pltpu.VMEM((nrows_w,), jnp.int32),
            pltpu.VMEM((_SC_NBUF, _SC_CH, D), jnp.float32),
            pltpu.SemaphoreType.DMA((_SC_NBUF,)),
            pltpu.SemaphoreType.DMA((_SC_NBUF,)),
        ],
        compiler_params=pltpu.CompilerParams(use_tc_tiling_on_sc=False),
    )(g, idx_flat)


def _gru_body(l0, l1, l2, l3, l4, tip_ref, gin, bin_, w_ih0, w_hh0, bih0, bhh0,
              w_ih1, w_hh1, bih1, bhh1, gout, bout,
              pw1, pb1, pw2, pb2, pw3, pb3, ftp_ref,
              ppn, g1n, bb1n, whn, *rest):
    lrefs = (l0, l1, l2, l3, l4)
    xs = [_ln(lrefs[t][...] + tip_ref[t], gin[...], bin_[...]) for t in range(TIN)]
    for (wi, wh, bi, bh) in ((w_ih0, w_hh0, bih0, bhh0), (w_ih1, w_hh1, bih1, bhh1)):
        h = jnp.zeros((TILE, D), jnp.float32)
        ys = []
        for t in range(TIN):
            gi = jnp.dot(xs[t], wi[...], preferred_element_type=jnp.float32) + bi[...]
            gh = jnp.dot(h, wh[...], preferred_element_type=jnp.float32) + bh[...]
            r = jax.nn.sigmoid(gi[:, :D] + gh[:, :D])
            z = jax.nn.sigmoid(gi[:, D:2 * D] + gh[:, D:2 * D])
            nn = jnp.tanh(gi[:, 2 * D:] + r * gh[:, 2 * D:])
            h = (1.0 - z) * nn + z * h
            ys.append(h)
        xs = ys
    seq = [_ln(y, gout[...], bout[...]) for y in xs]
    flat = jnp.concatenate(seq, axis=-1)
    p = _gelu(jnp.dot(flat, pw1[...], preferred_element_type=jnp.float32) + pb1[...])
    p = _gelu(jnp.dot(p, pw2[...], preferred_element_type=jnp.float32) + pb2[...])
    p = jnp.dot(p, pw3[...], preferred_element_type=jnp.float32) + pb3[...]
    orefs = rest[:TOUT]
    hrefs = rest[TOUT:2 * TOUT]
    grefs = rest[2 * TOUT:]
    for s in range(TOUT):
        fut = p[:, s * D:(s + 1) * D] + ftp_ref[s]
        orefs[s][...] = fut
        hh = _ln(fut, g1n[...], bb1n[...])
        hrefs[s][...] = hh
        grefs[s][...] = (jnp.dot(hh, whn[...], preferred_element_type=jnp.float32)
                         + ppn[...])


def _temporal(lats, tip, ftp, wp, pp, bp):
    row_spec = pl.BlockSpec((TILE, D), lambda i: (i, 0))
    args = (list(lats) + [tip] + list(wp) + [ftp]
            + [pp, bp['ln1g'], bp['ln1b'], bp['wh_t']])
    in_specs = [row_spec] * TIN + [_full((TIN, D))]
    in_specs += [_full(w.shape) for w in wp]
    in_specs += [_full((TOUT, D))]
    in_specs += [row_spec, _full((D,)), _full((D,)), _full((D, D))]
    return pl.pallas_call(
        _gru_body,
        grid=(NT,),
        in_specs=in_specs,
        out_specs=[row_spec] * (3 * TOUT),
        out_shape=[jax.ShapeDtypeStruct((N, D), jnp.float32)] * (3 * TOUT),
    )(*args)


def _dec_body(x0, x1, x2, x3, x4, v0_ref, w1, b1, w2, b2, w3, b3, o_ref):
    acc = v0_ref[...]
    xrefs = (x0, x1, x2, x3, x4)
    for s in range(TOUT):
        y = _gelu(jnp.dot(xrefs[s][...], w1[...], preferred_element_type=jnp.float32) + b1[...])
        y = _gelu(jnp.dot(y, w2[...], preferred_element_type=jnp.float32) + b2[...])
        y = jnp.dot(y, w3[...], preferred_element_type=jnp.float32) + b3[...]
        acc = acc + y
        o_ref[s] = acc


def _decoder(xs, v0, wp):
    w1, b1, w2, b2, w3, b3 = wp
    row_spec = pl.BlockSpec((TILE, D), lambda i: (i, 0))
    return pl.pallas_call(
        _dec_body,
        grid=(NT,),
        in_specs=[row_spec] * TOUT + [
            pl.BlockSpec((TILE, 3), lambda i: (i, 0)),
            _full(w1.shape), _full(b1.shape), _full(w2.shape),
            _full(b2.shape), _full(w3.shape), _full(b3.shape),
        ],
        out_specs=pl.BlockSpec((TOUT, TILE, 3), lambda i: (0, i, 0)),
        out_shape=jax.ShapeDtypeStruct((TOUT, N, 3), jnp.float32),
    )(*xs, v0, w1, b1, w2, b2, w3, b3)


def _pp_body(pos_ref, wr_ref, o_ref):
    o_ref[...] = jnp.dot(pos_ref[...], wr_ref[...], preferred_element_type=jnp.float32)


def _pos_proj(pos, wr_all):
    nb = wr_all.shape[1]
    return pl.pallas_call(
        _pp_body,
        grid=(NT,),
        in_specs=[pl.BlockSpec((TILE, 3), lambda i: (i, 0)), _full((3, nb))],
        out_specs=pl.BlockSpec((TILE, nb), lambda i: (i, 0)),
        out_shape=jax.ShapeDtypeStruct((N, nb), jnp.float32),
    )(pos, wr_all)


def _prep_block(p):
    wmsg = p['Wmsg']
    basis = p['basis']
    kb = basis.reshape(MODES, HEADS, DH)
    kbd = jax.scipy.linalg.block_diag(*[kb[:, h, :].T for h in range(HEADS)])
    vb = (basis @ p['Wv'].T).reshape(MODES, HEADS, DH)
    vbd = jax.scipy.linalg.block_diag(*[vb[:, h, :] for h in range(HEADS)])
    den = jnp.kron(jnp.eye(HEADS, dtype=jnp.float32), jnp.ones((MODES, 1), jnp.float32))
    exp_m = jnp.kron(jnp.eye(HEADS, dtype=jnp.float32), jnp.ones((1, DH), jnp.float32))
    (f1w, f1b), (f2w, f2b) = p['ffn']
    return {
        'ln1g': p['ln1'][0], 'ln1b': p['ln1'][1],
        'ln2g': p['ln2'][0], 'ln2b': p['ln2'][1],
        'ln3g': p['ln3'][0], 'ln3b': p['ln3'][1],
        'wh_t': wmsg[:, :D].T, 'wr_t': wmsg[:, D:].T, 'bmsg': p['bmsg'],
        'w1_t': p['Wupd'][:, :D].T, 'w2_t': p['Wupd'][:, D:].T, 'bupd': p['bupd'],
        'wqkbd': ((p['Wq'].T @ kbd) / math.sqrt(DH)).astype(jnp.bfloat16),
        'vbd': vbd.astype(jnp.bfloat16),
        'den': den.astype(jnp.bfloat16), 'exp_m': exp_m,
        'wf1_t': f1w.T, 'bf1': f1b, 'wf2_t': f2w.T, 'bf2': f2b,
    }


def _mlp_t(params):
    out = []
    for (w, b) in params:
        out.append(w.T)
        out.append(b)
    return out


def _mlp_jax(params, x):
    n = len(params)
    for i, (w, b) in enumerate(params):
        x = x @ w.T + b
        if i < n - 1:
            x = jax.nn.gelu(x)
    return x


def _fourier(t):
    freqs = jnp.pi * (2.0 ** jnp.arange(NF))
    a = t[..., None] * freqs
    return jnp.concatenate([t[..., None], jnp.sin(a), jnp.cos(a)], -1)


def _run_chain(x, h, g, blocks, pps, knn_flat):
    b0, b1 = blocks
    pp0, pp1 = pps
    gath = _gather_rows(g, knn_flat)
    x, h, g = _stage_b(gath, x, h, pp0, b0, tail=(pp1, b1))
    gath = _gather_rows(g, knn_flat)
    return _stage_b(gath, x, h, pp1, b1)[0]


def kernel(t, pos, idcs_airfoil, velocity_in, wall_distance, surface_frame,
           knn_indices, params):
    temb = _fourier(t)
    in_emb = temb[:, :TIN]
    out_emb = temb[:, TIN:]
    tip = _mlp_jax(params['temporal_input_proj'], in_emb)[0]
    ftp = _mlp_jax(params['future_time_proj'], out_emb)[0]

    pos2 = pos[0]
    mask = jnp.zeros((N,), jnp.float32).at[idcs_airfoil[0]].set(1.0)
    wall = jnp.log1p(wall_distance[0])[:, None]
    sf = surface_frame[0]

    rest = jnp.concatenate([wall, mask[:, None], sf], -1)
    feats = [jnp.concatenate([
        pos2,
        velocity_in[0, s],
        jnp.broadcast_to(in_emb[0, s][None, :], (N, TF)),
        rest,
    ], -1) for s in range(TIN)]

    enc_blocks = [_prep_block(p) for p in params['encoder_blocks']]
    fc_blocks = [_prep_block(p) for p in params['forecast_blocks']]

    wr_all = jnp.concatenate([bp['wr_t'] for bp in enc_blocks + fc_blocks], axis=1)
    pp_all = _pos_proj(pos2, wr_all)
    pps = [pp_all[:, i * D:(i + 1) * D] for i in range(4)]

    knn_flat = knn_indices[0].astype(jnp.int32).reshape(N * K)

    enc_w = _mlp_t(params['frame_encoder'])
    lats = []
    for s in range(TIN):
        x0, h0, g0 = _encoder(feats[s], enc_w, pps[0], enc_blocks[0])
        lats.append(_run_chain(x0, h0, g0, enc_blocks, pps[:2], knn_flat))

    gru = params['gru']
    wp = [params['temporal_input_norm'][0], params['temporal_input_norm'][1],
          gru[0]['Wih'].T, gru[0]['Whh'].T, gru[0]['bih'], gru[0]['bhh'],
          gru[1]['Wih'].T, gru[1]['Whh'].T, gru[1]['bih'], gru[1]['bhh'],
          params['temporal_output_norm'][0], params['temporal_output_norm'][1]]
    wp += _mlp_t(params['temporal_predictor'])
    tout = _temporal(lats, tip, ftp, wp, pps[2], fc_blocks[0])
    futs, fhs, fgs = tout[:TOUT], tout[TOUT:2 * TOUT], tout[2 * TOUT:]

    xfs = [_run_chain(f, hh, gg, fc_blocks, pps[2:], knn_flat)
           for f, hh, gg in zip(futs, fhs, fgs)]

    v0 = velocity_in[0, -1]
    outs = _decoder(xfs, v0, _mlp_t(params['decoder']))
    return outs[None]


# speedup vs baseline: 16.4605x; 1.0579x over previous
import functools
import math

import jax
import jax.numpy as jnp
from jax import lax
from jax.experimental import pallas as pl
from jax.experimental.pallas import tpu as pltpu
from jax.experimental.pallas import tpu_sc as plsc

B = 1
N = 8192
K = 16
D = 128
MODES = 256
HEADS = 8
DH = D // HEADS
TIN = 5
TOUT = 5
NF = 16
TF = 2 * NF + 1
TILE = 512
NT = N // TILE


def _ln(x, g, b):
    m = x.mean(-1, keepdims=True)
    v = ((x - m) ** 2).mean(-1, keepdims=True)
    return (x - m) * lax.rsqrt(v + 1e-5) * g + b


def _gelu(x):
    c = math.sqrt(2.0 / math.pi)
    return 0.5 * x * (1.0 + jnp.tanh(c * (x + 0.044715 * (x ** 3))))


def _full(spec_shape):
    nd = len(spec_shape)
    return pl.BlockSpec(spec_shape, lambda i, _nd=nd: (0,) * _nd)


def _enc_body(f_ref, w1, b1, w2, b2, w3, b3, pp_ref, g1, bb1, wh,
              o_ref, h_ref, g_ref):
    x = _gelu(jnp.dot(f_ref[...], w1[...], preferred_element_type=jnp.float32) + b1[...])
    x = _gelu(jnp.dot(x, w2[...], preferred_element_type=jnp.float32) + b2[...])
    x = jnp.dot(x, w3[...], preferred_element_type=jnp.float32) + b3[...]
    o_ref[...] = x
    h = _ln(x, g1[...], bb1[...])
    h_ref[...] = h
    g_ref[...] = jnp.dot(h, wh[...], preferred_element_type=jnp.float32) + pp_ref[...]


def _encoder(feats, wp, pp, bp):
    w1, b1, w2, b2, w3, b3 = wp
    row_spec = pl.BlockSpec((TILE, D), lambda i: (i, 0))
    return pl.pallas_call(
        _enc_body,
        grid=(NT,),
        in_specs=[
            pl.BlockSpec((TILE, feats.shape[1]), lambda i: (i, 0)),
            _full(w1.shape), _full(b1.shape), _full(w2.shape),
            _full(b2.shape), _full(w3.shape), _full(b3.shape),
            row_spec, _full((D,)), _full((D,)), _full((D, D)),
        ],
        out_specs=[row_spec] * 3,
        out_shape=[jax.ShapeDtypeStruct((N, D), jnp.float32)] * 3,
    )(feats, w1, b1, w2, b2, w3, b3, pp, bp['ln1g'], bp['ln1b'], bp['wh_t'])


def _stage_a_body(x_ref, pp_ref, g1, bb1, wh, h_ref, g_ref):
    h = _ln(x_ref[...], g1[...], bb1[...])
    h_ref[...] = h
    g_ref[...] = jnp.dot(h, wh[...], preferred_element_type=jnp.float32) + pp_ref[...]


def _stage_a(x, pp, bp):
    return pl.pallas_call(
        _stage_a_body,
        grid=(NT,),
        in_specs=[
            pl.BlockSpec((TILE, D), lambda i: (i, 0)),
            pl.BlockSpec((TILE, D), lambda i: (i, 0)),
            _full((D,)), _full((D,)), _full((D, D)),
        ],
        out_specs=[
            pl.BlockSpec((TILE, D), lambda i: (i, 0)),
            pl.BlockSpec((TILE, D), lambda i: (i, 0)),
        ],
        out_shape=[
            jax.ShapeDtypeStruct((N, D), jnp.float32),
            jax.ShapeDtypeStruct((N, D), jnp.float32),
        ],
    )(x, pp, bp['ln1g'], bp['ln1b'], bp['wh_t'])


def _stage_b_body(gath_ref, x_ref, h_ref, pp_ref,
                  bmsg, w1, w2, bupd, g2, bb2, wqkbd, vbd, den, exp_m,
                  g3, bb3, wf1, bf1, wf2, bf2, *rest):
    gath = gath_ref[...].reshape(TILE, K, D)
    c = pp_ref[...] - bmsg[...]
    msg = _gelu(gath - c[:, None, :])
    agg = jnp.mean(msg, axis=1)
    x1 = (x_ref[...]
          + jnp.dot(h_ref[...], w1[...], preferred_element_type=jnp.float32)
          + jnp.dot(agg, w2[...], preferred_element_type=jnp.float32)
          + bupd[...])
    h2 = _ln(x1, g2[...], bb2[...])
    s = jnp.dot(h2.astype(jnp.bfloat16), wqkbd[...],
                preferred_element_type=jnp.float32)
    e = jnp.exp(s)
    eb = e.astype(jnp.bfloat16)
    denom = jnp.dot(eb, den[...], preferred_element_type=jnp.float32)
    dfull = jnp.dot(denom, exp_m[...], preferred_element_type=jnp.float32)
    numer = jnp.dot(eb, vbd[...], preferred_element_type=jnp.float32)
    x2 = x1 + numer / dfull
    h3 = _ln(x2, g3[...], bb3[...])
    f = _gelu(jnp.dot(h3, wf1[...], preferred_element_type=jnp.float32) + bf1[...])
    x3 = x2 + jnp.dot(f, wf2[...], preferred_element_type=jnp.float32) + bf2[...]
    if len(rest) == 1:
        rest[0][...] = x3
    else:
        ppn, g1n, bb1n, whn, o_ref, hn_ref, gn_ref = rest
        o_ref[...] = x3
        hn = _ln(x3, g1n[...], bb1n[...])
        hn_ref[...] = hn
        gn_ref[...] = (jnp.dot(hn, whn[...], preferred_element_type=jnp.float32)
                       + ppn[...])


def _stage_b(gath, x, h, pp, bp, tail=None):
    row_spec = pl.BlockSpec((TILE, D), lambda i: (i, 0))
    in_specs = [
        pl.BlockSpec((TILE * K, D), lambda i: (i, 0)),
        row_spec, row_spec, row_spec,
        _full((D,)), _full((D, D)), _full((D, D)), _full((D,)),
        _full((D,)), _full((D,)),
        _full((D, HEADS * MODES)), _full((HEADS * MODES, D)),
        _full((HEADS * MODES, HEADS)), _full((HEADS, D)),
        _full((D,)), _full((D,)),
        _full((D, 2 * D)), _full((2 * D,)), _full((2 * D, D)), _full((D,)),
    ]
    args = [gath, x, h, pp,
            bp['bmsg'], bp['w1_t'], bp['w2_t'], bp['bupd'],
            bp['ln2g'], bp['ln2b'], bp['wqkbd'], bp['vbd'], bp['den'],
            bp['exp_m'],
            bp['ln3g'], bp['ln3b'], bp['wf1_t'], bp['bf1'], bp['wf2_t'],
            bp['bf2']]
    if tail is None:
        n_out = 1
    else:
        ppn, bpn = tail
        in_specs += [row_spec, _full((D,)), _full((D,)), _full((D, D))]
        args += [ppn, bpn['ln1g'], bpn['ln1b'], bpn['wh_t']]
        n_out = 3
    return pl.pallas_call(
        _stage_b_body,
        grid=(NT,),
        in_specs=in_specs,
        out_specs=[row_spec] * n_out,
        out_shape=[jax.ShapeDtypeStruct((N, D), jnp.float32)] * n_out,
    )(*args)


_SC_CH = 128
_SC_NBUF = 4


def _sc_gather_body(g_hbm, idx_hbm, out_hbm, idx_v, rows_v, sems_g, sems_o,
                    nrows_w):
    wid = lax.axis_index("s") * 2 + lax.axis_index("c")
    base = wid * nrows_w
    nch = nrows_w // _SC_CH
    pltpu.sync_copy(idx_hbm.at[pl.ds(base, nrows_w)], idx_v)

    def gstart(i, b):
        pltpu.async_copy(g_hbm.at[idx_v.at[pl.ds(i * _SC_CH, _SC_CH)]],
                         rows_v.at[b], sems_g.at[b])

    def gwait(i, b):
        pltpu.make_async_copy(g_hbm.at[idx_v.at[pl.ds(i * _SC_CH, _SC_CH)]],
                              rows_v.at[b], sems_g.at[b]).wait()

    def ostart(i, b):
        pltpu.async_copy(rows_v.at[b],
                         out_hbm.at[pl.ds(base + i * _SC_CH, _SC_CH)],
                         sems_o.at[b])

    def owait(i, b):
        pltpu.make_async_copy(rows_v.at[b],
                              out_hbm.at[pl.ds(base + i * _SC_CH, _SC_CH)],
                              sems_o.at[b]).wait()

    for b in range(_SC_NBUF):
        gstart(b, b)

    @pl.loop(0, nch, step=_SC_NBUF)
    def _(ci):
        for b in range(_SC_NBUF):
            gwait(ci + b, b)
            ostart(ci + b, b)
        for b in range(_SC_NBUF):
            @pl.when(ci + b + _SC_NBUF < nch)
            def _():
                owait(ci + b, b)
                gstart(ci + b + _SC_NBUF, b)

    for b in range(_SC_NBUF):
        owait(nch - _SC_NBUF + b, b)


def _gather_rows(g, idx_flat):
    nw = 32
    nrows = idx_flat.shape[0]
    nrows_w = nrows // nw
    mesh = plsc.VectorSubcoreMesh(core_axis_name="c", subcore_axis_name="s")
    body = functools.partial(_sc_gather_body, nrows_w=nrows_w)
    return pl.kernel(
        body,
        out_type=jax.ShapeDtypeStruct((nrows, D), jnp.float32),
        mesh=mesh,
        scratch_types=[
            pltpu.VMEM((nrows_w,), jnp.int32),
            pltpu.VMEM((_SC_NBUF, _SC_CH, D), jnp.float32),
            pltpu.SemaphoreType.DMA((_SC_NBUF,)),
            pltpu.SemaphoreType.DMA((_SC_NBUF,)),
        ],
        compiler_params=pltpu.CompilerParams(use_tc_tiling_on_sc=False),
    )(g, idx_flat)


def _gru_body(l0, l1, l2, l3, l4, tip_ref, gin, bin_, w_ih0, w_hh0, bih0, bhh0,
              w_ih1, w_hh1, bih1, bhh1, gout, bout,
              pw1, pb1, pw2, pb2, pw3, pb3, ftp_ref,
              ppn, g1n, bb1n, whn, *rest):
    lrefs = (l0, l1, l2, l3, l4)
    xs = [_ln(lrefs[t][...] + tip_ref[t], gin[...], bin_[...]) for t in range(TIN)]
    for (wi, wh, bi, bh) in ((w_ih0, w_hh0, bih0, bhh0), (w_ih1, w_hh1, bih1, bhh1)):
        h = jnp.zeros((TILE, D), jnp.float32)
        ys = []
        for t in range(TIN):
            gi = jnp.dot(xs[t], wi[...], preferred_element_type=jnp.float32) + bi[...]
            gh = jnp.dot(h, wh[...], preferred_element_type=jnp.float32) + bh[...]
            r = jax.nn.sigmoid(gi[:, :D] + gh[:, :D])
            z = jax.nn.sigmoid(gi[:, D:2 * D] + gh[:, D:2 * D])
            nn = jnp.tanh(gi[:, 2 * D:] + r * gh[:, 2 * D:])
            h = (1.0 - z) * nn + z * h
            ys.append(h)
        xs = ys
    seq = [_ln(y, gout[...], bout[...]) for y in xs]
    flat = jnp.concatenate(seq, axis=-1)
    p = _gelu(jnp.dot(flat, pw1[...], preferred_element_type=jnp.float32) + pb1[...])
    p = _gelu(jnp.dot(p, pw2[...], preferred_element_type=jnp.float32) + pb2[...])
    p = jnp.dot(p, pw3[...], preferred_element_type=jnp.float32) + pb3[...]
    orefs = rest[:TOUT]
    hrefs = rest[TOUT:2 * TOUT]
    grefs = rest[2 * TOUT:]
    for s in range(TOUT):
        fut = p[:, s * D:(s + 1) * D] + ftp_ref[s]
        orefs[s][...] = fut
        hh = _ln(fut, g1n[...], bb1n[...])
        hrefs[s][...] = hh
        grefs[s][...] = (jnp.dot(hh, whn[...], preferred_element_type=jnp.float32)
                         + ppn[...])


def _temporal(lats, tip, ftp, wp, pp, bp):
    row_spec = pl.BlockSpec((TILE, D), lambda i: (i, 0))
    args = (list(lats) + [tip] + list(wp) + [ftp]
            + [pp, bp['ln1g'], bp['ln1b'], bp['wh_t']])
    in_specs = [row_spec] * TIN + [_full((TIN, D))]
    in_specs += [_full(w.shape) for w in wp]
    in_specs += [_full((TOUT, D))]
    in_specs += [row_spec, _full((D,)), _full((D,)), _full((D, D))]
    return pl.pallas_call(
        _gru_body,
        grid=(NT,),
        in_specs=in_specs,
        out_specs=[row_spec] * (3 * TOUT),
        out_shape=[jax.ShapeDtypeStruct((N, D), jnp.float32)] * (3 * TOUT),
    )(*args)


def _dec_body(x0, x1, x2, x3, x4, v0_ref, w1, b1, w2, b2, w3, b3, o_ref):
    acc = v0_ref[...]
    xrefs = (x0, x1, x2, x3, x4)
    for s in range(TOUT):
        y = _gelu(jnp.dot(xrefs[s][...], w1[...], preferred_element_type=jnp.float32) + b1[...])
        y = _gelu(jnp.dot(y, w2[...], preferred_element_type=jnp.float32) + b2[...])
        y = jnp.dot(y, w3[...], preferred_element_type=jnp.float32) + b3[...]
        acc = acc + y
        o_ref[s] = acc


def _decoder(xs, v0, wp):
    w1, b1, w2, b2, w3, b3 = wp
    row_spec = pl.BlockSpec((TILE, D), lambda i: (i, 0))
    return pl.pallas_call(
        _dec_body,
        grid=(NT,),
        in_specs=[row_spec] * TOUT + [
            pl.BlockSpec((TILE, 3), lambda i: (i, 0)),
            _full(w1.shape), _full(b1.shape), _full(w2.shape),
            _full(b2.shape), _full(w3.shape), _full(b3.shape),
        ],
        out_specs=pl.BlockSpec((TOUT, TILE, 3), lambda i: (0, i, 0)),
        out_shape=jax.ShapeDtypeStruct((TOUT, N, 3), jnp.float32),
    )(*xs, v0, w1, b1, w2, b2, w3, b3)


def _pp_body(pos_ref, wr_ref, o_ref):
    o_ref[...] = jnp.dot(pos_ref[...], wr_ref[...], preferred_element_type=jnp.float32)


def _pos_proj(pos, wr_all):
    nb = wr_all.shape[1]
    return pl.pallas_call(
        _pp_body,
        grid=(NT,),
        in_specs=[pl.BlockSpec((TILE, 3), lambda i: (i, 0)), _full((3, nb))],
        out_specs=pl.BlockSpec((TILE, nb), lambda i: (i, 0)),
        out_shape=jax.ShapeDtypeStruct((N, nb), jnp.float32),
    )(pos, wr_all)


def _prep_block(p):
    wmsg = p['Wmsg']
    basis = p['basis']
    kb = basis.reshape(MODES, HEADS, DH)
    kbd = jax.scipy.linalg.block_diag(*[kb[:, h, :].T for h in range(HEADS)])
    vb = (basis @ p['Wv'].T).reshape(MODES, HEADS, DH)
    vbd = jax.scipy.linalg.block_diag(*[vb[:, h, :] for h in range(HEADS)])
    den = jnp.kron(jnp.eye(HEADS, dtype=jnp.float32), jnp.ones((MODES, 1), jnp.float32))
    exp_m = jnp.kron(jnp.eye(HEADS, dtype=jnp.float32), jnp.ones((1, DH), jnp.float32))
    (f1w, f1b), (f2w, f2b) = p['ffn']
    return {
        'ln1g': p['ln1'][0], 'ln1b': p['ln1'][1],
        'ln2g': p['ln2'][0], 'ln2b': p['ln2'][1],
        'ln3g': p['ln3'][0], 'ln3b': p['ln3'][1],
        'wh_t': wmsg[:, :D].T, 'wr_t': wmsg[:, D:].T, 'bmsg': p['bmsg'],
        'w1_t': p['Wupd'][:, :D].T, 'w2_t': p['Wupd'][:, D:].T, 'bupd': p['bupd'],
        'wqkbd': ((p['Wq'].T @ kbd) / math.sqrt(DH)).astype(jnp.bfloat16),
        'vbd': vbd.astype(jnp.bfloat16),
        'den': den.astype(jnp.bfloat16), 'exp_m': exp_m,
        'wf1_t': f1w.T, 'bf1': f1b, 'wf2_t': f2w.T, 'bf2': f2b,
    }


def _mlp_t(params):
    out = []
    for (w, b) in params:
        out.append(w.T)
        out.append(b)
    return out


def _mlp_jax(params, x):
    n = len(params)
    for i, (w, b) in enumerate(params):
        x = x @ w.T + b
        if i < n - 1:
            x = jax.nn.gelu(x)
    return x


def _fourier(t):
    freqs = jnp.pi * (2.0 ** jnp.arange(NF))
    a = t[..., None] * freqs
    return jnp.concatenate([t[..., None], jnp.sin(a), jnp.cos(a)], -1)


def _run_chain(x, h, g, blocks, pps, knn_flat):
    b0, b1 = blocks
    pp0, pp1 = pps
    gath = _gather_rows(g, knn_flat)
    x, h, g = _stage_b(gath, x, h, pp0, b0, tail=(pp1, b1))
    gath = _gather_rows(g, knn_flat)
    return _stage_b(gath, x, h, pp1, b1)[0]


def kernel(t, pos, idcs_airfoil, velocity_in, wall_distance, surface_frame,
           knn_indices, params):
    temb = _fourier(t)
    in_emb = temb[:, :TIN]
    out_emb = temb[:, TIN:]
    tip = _mlp_jax(params['temporal_input_proj'], in_emb)[0]
    ftp = _mlp_jax(params['future_time_proj'], out_emb)[0]

    pos2 = pos[0]
    mask = jnp.zeros((N,), jnp.float32).at[idcs_airfoil[0]].set(1.0)
    wall = jnp.log1p(wall_distance[0])[:, None]
    sf = surface_frame[0]

    rest = jnp.concatenate([wall, mask[:, None], sf], -1)
    feats = [jnp.concatenate([
        pos2,
        velocity_in[0, s],
        jnp.broadcast_to(in_emb[0, s][None, :], (N, TF)),
        rest,
    ], -1) for s in range(TIN)]

    enc_blocks = [_prep_block(p) for p in params['encoder_blocks']]
    fc_blocks = [_prep_block(p) for p in params['forecast_blocks']]

    wr_all = jnp.concatenate([bp['wr_t'] for bp in enc_blocks + fc_blocks], axis=1)
    pp_all = _pos_proj(pos2, wr_all)
    pps = [pp_all[:, i * D:(i + 1) * D] for i in range(4)]

    knn_flat = knn_indices[0].astype(jnp.int32).reshape(N * K)

    enc_w = _mlp_t(params['frame_encoder'])
    lats = []
    for s in range(TIN):
        x0, h0, g0 = _encoder(feats[s], enc_w, pps[0], enc_blocks[0])
        lats.append(_run_chain(x0, h0, g0, enc_blocks, pps[:2], knn_flat))

    gru = params['gru']
    wp = [params['temporal_input_norm'][0], params['temporal_input_norm'][1],
          gru[0]['Wih'].T, gru[0]['Whh'].T, gru[0]['bih'], gru[0]['bhh'],
          gru[1]['Wih'].T, gru[1]['Whh'].T, gru[1]['bih'], gru[1]['bhh'],
          params['temporal_output_norm'][0], params['temporal_output_norm'][1]]
    wp += _mlp_t(params['temporal_predictor'])
    tout = _temporal(lats, tip, ftp, wp, pps[2], fc_blocks[0])
    futs, fhs, fgs = tout[:TOUT], tout[TOUT:2 * TOUT], tout[2 * TOUT:]

    xfs = [_run_chain(f, hh, gg, fc_blocks, pps[2:], knn_flat)
           for f, hh, gg in zip(futs, fhs, fgs)]

    v0 = velocity_in[0, -1]
    outs = _decoder(xfs, v0, _mlp_t(params['decoder']))
    return outs[None]


# speedup vs baseline: 17.1621x; 1.0426x over previous
import functools
import math

import jax
import jax.numpy as jnp
from jax import lax
from jax.experimental import pallas as pl
from jax.experimental.pallas import tpu as pltpu
from jax.experimental.pallas import tpu_sc as plsc

B = 1
N = 8192
K = 16
D = 128
MODES = 256
HEADS = 8
DH = D // HEADS
TIN = 5
TOUT = 5
NF = 16
TF = 2 * NF + 1
TILE = 1024
NT = N // TILE


def _ln(x, g, b):
    m = x.mean(-1, keepdims=True)
    v = ((x - m) ** 2).mean(-1, keepdims=True)
    return (x - m) * lax.rsqrt(v + 1e-5) * g + b


def _gelu(x):
    c = math.sqrt(2.0 / math.pi)
    return 0.5 * x * (1.0 + jnp.tanh(c * (x + 0.044715 * (x ** 3))))


def _full(spec_shape):
    nd = len(spec_shape)
    return pl.BlockSpec(spec_shape, lambda i, _nd=nd: (0,) * _nd)


def _enc_body(f_ref, w1, b1, w2, b2, w3, b3, pp_ref, g1, bb1, wh,
              o_ref, h_ref, g_ref):
    x = _gelu(jnp.dot(f_ref[...], w1[...], preferred_element_type=jnp.float32) + b1[...])
    x = _gelu(jnp.dot(x, w2[...], preferred_element_type=jnp.float32) + b2[...])
    x = jnp.dot(x, w3[...], preferred_element_type=jnp.float32) + b3[...]
    o_ref[...] = x
    h = _ln(x, g1[...], bb1[...])
    h_ref[...] = h
    g_ref[...] = jnp.dot(h, wh[...], preferred_element_type=jnp.float32) + pp_ref[...]


def _encoder(feats, wp, pp, bp):
    w1, b1, w2, b2, w3, b3 = wp
    row_spec = pl.BlockSpec((TILE, D), lambda i: (i, 0))
    return pl.pallas_call(
        _enc_body,
        grid=(NT,),
        in_specs=[
            pl.BlockSpec((TILE, feats.shape[1]), lambda i: (i, 0)),
            _full(w1.shape), _full(b1.shape), _full(w2.shape),
            _full(b2.shape), _full(w3.shape), _full(b3.shape),
            row_spec, _full((D,)), _full((D,)), _full((D, D)),
        ],
        out_specs=[row_spec] * 3,
        out_shape=[jax.ShapeDtypeStruct((N, D), jnp.float32)] * 3,
    )(feats, w1, b1, w2, b2, w3, b3, pp, bp['ln1g'], bp['ln1b'], bp['wh_t'])


def _stage_a_body(x_ref, pp_ref, g1, bb1, wh, h_ref, g_ref):
    h = _ln(x_ref[...], g1[...], bb1[...])
    h_ref[...] = h
    g_ref[...] = jnp.dot(h, wh[...], preferred_element_type=jnp.float32) + pp_ref[...]


def _stage_a(x, pp, bp):
    return pl.pallas_call(
        _stage_a_body,
        grid=(NT,),
        in_specs=[
            pl.BlockSpec((TILE, D), lambda i: (i, 0)),
            pl.BlockSpec((TILE, D), lambda i: (i, 0)),
            _full((D,)), _full((D,)), _full((D, D)),
        ],
        out_specs=[
            pl.BlockSpec((TILE, D), lambda i: (i, 0)),
            pl.BlockSpec((TILE, D), lambda i: (i, 0)),
        ],
        out_shape=[
            jax.ShapeDtypeStruct((N, D), jnp.float32),
            jax.ShapeDtypeStruct((N, D), jnp.float32),
        ],
    )(x, pp, bp['ln1g'], bp['ln1b'], bp['wh_t'])


def _stage_b_body(gath_ref, x_ref, h_ref, pp_ref,
                  bmsg, w1, w2, bupd, g2, bb2, wqkbd, vbd, den, exp_m,
                  g3, bb3, wf1, bf1, wf2, bf2, *rest):
    gath = gath_ref[...].reshape(TILE, K, D)
    c = pp_ref[...] - bmsg[...]
    msg = _gelu(gath - c[:, None, :])
    agg = jnp.mean(msg, axis=1)
    x1 = (x_ref[...]
          + jnp.dot(h_ref[...], w1[...], preferred_element_type=jnp.float32)
          + jnp.dot(agg, w2[...], preferred_element_type=jnp.float32)
          + bupd[...])
    h2 = _ln(x1, g2[...], bb2[...])
    s = jnp.dot(h2.astype(jnp.bfloat16), wqkbd[...],
                preferred_element_type=jnp.float32)
    e = jnp.exp(s)
    eb = e.astype(jnp.bfloat16)
    denom = jnp.dot(eb, den[...], preferred_element_type=jnp.float32)
    dfull = jnp.dot(denom, exp_m[...], preferred_element_type=jnp.float32)
    numer = jnp.dot(eb, vbd[...], preferred_element_type=jnp.float32)
    x2 = x1 + numer / dfull
    h3 = _ln(x2, g3[...], bb3[...])
    f = _gelu(jnp.dot(h3, wf1[...], preferred_element_type=jnp.float32) + bf1[...])
    x3 = x2 + jnp.dot(f, wf2[...], preferred_element_type=jnp.float32) + bf2[...]
    if len(rest) == 1:
        rest[0][...] = x3
    else:
        ppn, g1n, bb1n, whn, o_ref, hn_ref, gn_ref = rest
        o_ref[...] = x3
        hn = _ln(x3, g1n[...], bb1n[...])
        hn_ref[...] = hn
        gn_ref[...] = (jnp.dot(hn, whn[...], preferred_element_type=jnp.float32)
                       + ppn[...])


def _stage_b(gath, x, h, pp, bp, tail=None):
    row_spec = pl.BlockSpec((TILE, D), lambda i: (i, 0))
    in_specs = [
        pl.BlockSpec((TILE * K, D), lambda i: (i, 0)),
        row_spec, row_spec, row_spec,
        _full((D,)), _full((D, D)), _full((D, D)), _full((D,)),
        _full((D,)), _full((D,)),
        _full((D, HEADS * MODES)), _full((HEADS * MODES, D)),
        _full((HEADS * MODES, HEADS)), _full((HEADS, D)),
        _full((D,)), _full((D,)),
        _full((D, 2 * D)), _full((2 * D,)), _full((2 * D, D)), _full((D,)),
    ]
    args = [gath, x, h, pp,
            bp['bmsg'], bp['w1_t'], bp['w2_t'], bp['bupd'],
            bp['ln2g'], bp['ln2b'], bp['wqkbd'], bp['vbd'], bp['den'],
            bp['exp_m'],
            bp['ln3g'], bp['ln3b'], bp['wf1_t'], bp['bf1'], bp['wf2_t'],
            bp['bf2']]
    if tail is None:
        n_out = 1
    else:
        ppn, bpn = tail
        in_specs += [row_spec, _full((D,)), _full((D,)), _full((D, D))]
        args += [ppn, bpn['ln1g'], bpn['ln1b'], bpn['wh_t']]
        n_out = 3
    return pl.pallas_call(
        _stage_b_body,
        grid=(NT,),
        in_specs=in_specs,
        out_specs=[row_spec] * n_out,
        out_shape=[jax.ShapeDtypeStruct((N, D), jnp.float32)] * n_out,
    )(*args)


_SC_CH = 128
_SC_NBUF = 4


def _sc_gather_body(g_hbm, idx_hbm, out_hbm, idx_v, rows_v, sems_g, sems_o,
                    nrows_w):
    wid = lax.axis_index("s") * 2 + lax.axis_index("c")
    base = wid * nrows_w
    nch = nrows_w // _SC_CH
    pltpu.sync_copy(idx_hbm.at[pl.ds(base, nrows_w)], idx_v)

    def gstart(i, b):
        pltpu.async_copy(g_hbm.at[idx_v.at[pl.ds(i * _SC_CH, _SC_CH)]],
                         rows_v.at[b], sems_g.at[b])

    def gwait(i, b):
        pltpu.make_async_copy(g_hbm.at[idx_v.at[pl.ds(i * _SC_CH, _SC_CH)]],
                              rows_v.at[b], sems_g.at[b]).wait()

    def ostart(i, b):
        pltpu.async_copy(rows_v.at[b],
                         out_hbm.at[pl.ds(base + i * _SC_CH, _SC_CH)],
                         sems_o.at[b])

    def owait(i, b):
        pltpu.make_async_copy(rows_v.at[b],
                              out_hbm.at[pl.ds(base + i * _SC_CH, _SC_CH)],
                              sems_o.at[b]).wait()

    for b in range(_SC_NBUF):
        gstart(b, b)

    @pl.loop(0, nch, step=_SC_NBUF)
    def _(ci):
        for b in range(_SC_NBUF):
            gwait(ci + b, b)
            ostart(ci + b, b)
        for b in range(_SC_NBUF):
            @pl.when(ci + b + _SC_NBUF < nch)
            def _():
                owait(ci + b, b)
                gstart(ci + b + _SC_NBUF, b)

    for b in range(_SC_NBUF):
        owait(nch - _SC_NBUF + b, b)


def _gather_rows(g, idx_flat):
    nw = 32
    nrows = idx_flat.shape[0]
    nrows_w = nrows // nw
    mesh = plsc.VectorSubcoreMesh(core_axis_name="c", subcore_axis_name="s")
    body = functools.partial(_sc_gather_body, nrows_w=nrows_w)
    return pl.kernel(
        body,
        out_type=jax.ShapeDtypeStruct((nrows, D), jnp.float32),
        mesh=mesh,
        scratch_types=[
            pltpu.VMEM((nrows_w,), jnp.int32),
            pltpu.VMEM((_SC_NBUF, _SC_CH, D), jnp.float32),
            pltpu.SemaphoreType.DMA((_SC_NBUF,)),
            pltpu.SemaphoreType.DMA((_SC_NBUF,)),
        ],
        compiler_params=pltpu.CompilerParams(use_tc_tiling_on_sc=False),
    )(g, idx_flat)


def _gru_body(l0, l1, l2, l3, l4, tip_ref, gin, bin_, w_ih0, w_hh0, bih0, bhh0,
              w_ih1, w_hh1, bih1, bhh1, gout, bout,
              pw1, pb1, pw2, pb2, pw3, pb3, ftp_ref,
              ppn, g1n, bb1n, whn, *rest):
    lrefs = (l0, l1, l2, l3, l4)
    xs = [_ln(lrefs[t][...] + tip_ref[t], gin[...], bin_[...]) for t in range(TIN)]
    for (wi, wh, bi, bh) in ((w_ih0, w_hh0, bih0, bhh0), (w_ih1, w_hh1, bih1, bhh1)):
        h = jnp.zeros((TILE, D), jnp.float32)
        ys = []
        for t in range(TIN):
            gi = jnp.dot(xs[t], wi[...], preferred_element_type=jnp.float32) + bi[...]
            gh = jnp.dot(h, wh[...], preferred_element_type=jnp.float32) + bh[...]
            r = jax.nn.sigmoid(gi[:, :D] + gh[:, :D])
            z = jax.nn.sigmoid(gi[:, D:2 * D] + gh[:, D:2 * D])
            nn = jnp.tanh(gi[:, 2 * D:] + r * gh[:, 2 * D:])
            h = (1.0 - z) * nn + z * h
            ys.append(h)
        xs = ys
    seq = [_ln(y, gout[...], bout[...]) for y in xs]
    flat = jnp.concatenate(seq, axis=-1)
    p = _gelu(jnp.dot(flat, pw1[...], preferred_element_type=jnp.float32) + pb1[...])
    p = _gelu(jnp.dot(p, pw2[...], preferred_element_type=jnp.float32) + pb2[...])
    p = jnp.dot(p, pw3[...], preferred_element_type=jnp.float32) + pb3[...]
    orefs = rest[:TOUT]
    hrefs = rest[TOUT:2 * TOUT]
    grefs = rest[2 * TOUT:]
    for s in range(TOUT):
        fut = p[:, s * D:(s + 1) * D] + ftp_ref[s]
        orefs[s][...] = fut
        hh = _ln(fut, g1n[...], bb1n[...])
        hrefs[s][...] = hh
        grefs[s][...] = (jnp.dot(hh, whn[...], preferred_element_type=jnp.float32)
                         + ppn[...])


def _temporal(lats, tip, ftp, wp, pp, bp):
    row_spec = pl.BlockSpec((TILE, D), lambda i: (i, 0))
    args = (list(lats) + [tip] + list(wp) + [ftp]
            + [pp, bp['ln1g'], bp['ln1b'], bp['wh_t']])
    in_specs = [row_spec] * TIN + [_full((TIN, D))]
    in_specs += [_full(w.shape) for w in wp]
    in_specs += [_full((TOUT, D))]
    in_specs += [row_spec, _full((D,)), _full((D,)), _full((D, D))]
    return pl.pallas_call(
        _gru_body,
        grid=(NT,),
        in_specs=in_specs,
        out_specs=[row_spec] * (3 * TOUT),
        out_shape=[jax.ShapeDtypeStruct((N, D), jnp.float32)] * (3 * TOUT),
    )(*args)


def _dec_body(x0, x1, x2, x3, x4, v0_ref, w1, b1, w2, b2, w3, b3, o_ref):
    acc = v0_ref[...]
    xrefs = (x0, x1, x2, x3, x4)
    for s in range(TOUT):
        y = _gelu(jnp.dot(xrefs[s][...], w1[...], preferred_element_type=jnp.float32) + b1[...])
        y = _gelu(jnp.dot(y, w2[...], preferred_element_type=jnp.float32) + b2[...])
        y = jnp.dot(y, w3[...], preferred_element_type=jnp.float32) + b3[...]
        acc = acc + y
        o_ref[s] = acc


def _decoder(xs, v0, wp):
    w1, b1, w2, b2, w3, b3 = wp
    row_spec = pl.BlockSpec((TILE, D), lambda i: (i, 0))
    return pl.pallas_call(
        _dec_body,
        grid=(NT,),
        in_specs=[row_spec] * TOUT + [
            pl.BlockSpec((TILE, 3), lambda i: (i, 0)),
            _full(w1.shape), _full(b1.shape), _full(w2.shape),
            _full(b2.shape), _full(w3.shape), _full(b3.shape),
        ],
        out_specs=pl.BlockSpec((TOUT, TILE, 3), lambda i: (0, i, 0)),
        out_shape=jax.ShapeDtypeStruct((TOUT, N, 3), jnp.float32),
    )(*xs, v0, w1, b1, w2, b2, w3, b3)


def _pp_body(pos_ref, wr_ref, o_ref):
    o_ref[...] = jnp.dot(pos_ref[...], wr_ref[...], preferred_element_type=jnp.float32)


def _pos_proj(pos, wr_all):
    nb = wr_all.shape[1]
    return pl.pallas_call(
        _pp_body,
        grid=(NT,),
        in_specs=[pl.BlockSpec((TILE, 3), lambda i: (i, 0)), _full((3, nb))],
        out_specs=pl.BlockSpec((TILE, nb), lambda i: (i, 0)),
        out_shape=jax.ShapeDtypeStruct((N, nb), jnp.float32),
    )(pos, wr_all)


def _prep_block(p):
    wmsg = p['Wmsg']
    basis = p['basis']
    kb = basis.reshape(MODES, HEADS, DH)
    kbd = jax.scipy.linalg.block_diag(*[kb[:, h, :].T for h in range(HEADS)])
    vb = (basis @ p['Wv'].T).reshape(MODES, HEADS, DH)
    vbd = jax.scipy.linalg.block_diag(*[vb[:, h, :] for h in range(HEADS)])
    den = jnp.kron(jnp.eye(HEADS, dtype=jnp.float32), jnp.ones((MODES, 1), jnp.float32))
    exp_m = jnp.kron(jnp.eye(HEADS, dtype=jnp.float32), jnp.ones((1, DH), jnp.float32))
    (f1w, f1b), (f2w, f2b) = p['ffn']
    return {
        'ln1g': p['ln1'][0], 'ln1b': p['ln1'][1],
        'ln2g': p['ln2'][0], 'ln2b': p['ln2'][1],
        'ln3g': p['ln3'][0], 'ln3b': p['ln3'][1],
        'wh_t': wmsg[:, :D].T, 'wr_t': wmsg[:, D:].T, 'bmsg': p['bmsg'],
        'w1_t': p['Wupd'][:, :D].T, 'w2_t': p['Wupd'][:, D:].T, 'bupd': p['bupd'],
        'wqkbd': ((p['Wq'].T @ kbd) / math.sqrt(DH)).astype(jnp.bfloat16),
        'vbd': vbd.astype(jnp.bfloat16),
        'den': den.astype(jnp.bfloat16), 'exp_m': exp_m,
        'wf1_t': f1w.T, 'bf1': f1b, 'wf2_t': f2w.T, 'bf2': f2b,
    }


def _mlp_t(params):
    out = []
    for (w, b) in params:
        out.append(w.T)
        out.append(b)
    return out


def _mlp_jax(params, x):
    n = len(params)
    for i, (w, b) in enumerate(params):
        x = x @ w.T + b
        if i < n - 1:
            x = jax.nn.gelu(x)
    return x


def _fourier(t):
    freqs = jnp.pi * (2.0 ** jnp.arange(NF))
    a = t[..., None] * freqs
    return jnp.concatenate([t[..., None], jnp.sin(a), jnp.cos(a)], -1)


def _run_chain(x, h, g, blocks, pps, knn_flat):
    b0, b1 = blocks
    pp0, pp1 = pps
    gath = _gather_rows(g, knn_flat)
    x, h, g = _stage_b(gath, x, h, pp0, b0, tail=(pp1, b1))
    gath = _gather_rows(g, knn_flat)
    return _stage_b(gath, x, h, pp1, b1)[0]


def kernel(t, pos, idcs_airfoil, velocity_in, wall_distance, surface_frame,
           knn_indices, params):
    temb = _fourier(t)
    in_emb = temb[:, :TIN]
    out_emb = temb[:, TIN:]
    tip = _mlp_jax(params['temporal_input_proj'], in_emb)[0]
    ftp = _mlp_jax(params['future_time_proj'], out_emb)[0]

    pos2 = pos[0]
    mask = jnp.zeros((N,), jnp.float32).at[idcs_airfoil[0]].set(1.0)
    wall = jnp.log1p(wall_distance[0])[:, None]
    sf = surface_frame[0]

    rest = jnp.concatenate([wall, mask[:, None], sf], -1)
    feats = [jnp.concatenate([
        pos2,
        velocity_in[0, s],
        jnp.broadcast_to(in_emb[0, s][None, :], (N, TF)),
        rest,
    ], -1) for s in range(TIN)]

    enc_blocks = [_prep_block(p) for p in params['encoder_blocks']]
    fc_blocks = [_prep_block(p) for p in params['forecast_blocks']]

    wr_all = jnp.concatenate([bp['wr_t'] for bp in enc_blocks + fc_blocks], axis=1)
    pp_all = _pos_proj(pos2, wr_all)
    pps = [pp_all[:, i * D:(i + 1) * D] for i in range(4)]

    knn_flat = knn_indices[0].astype(jnp.int32).reshape(N * K)

    enc_w = _mlp_t(params['frame_encoder'])
    lats = []
    for s in range(TIN):
        x0, h0, g0 = _encoder(feats[s], enc_w, pps[0], enc_blocks[0])
        lats.append(_run_chain(x0, h0, g0, enc_blocks, pps[:2], knn_flat))

    gru = params['gru']
    wp = [params['temporal_input_norm'][0], params['temporal_input_norm'][1],
          gru[0]['Wih'].T, gru[0]['Whh'].T, gru[0]['bih'], gru[0]['bhh'],
          gru[1]['Wih'].T, gru[1]['Whh'].T, gru[1]['bih'], gru[1]['bhh'],
          params['temporal_output_norm'][0], params['temporal_output_norm'][1]]
    wp += _mlp_t(params['temporal_predictor'])
    tout = _temporal(lats, tip, ftp, wp, pps[2], fc_blocks[0])
    futs, fhs, fgs = tout[:TOUT], tout[TOUT:2 * TOUT], tout[2 * TOUT:]

    xfs = [_run_chain(f, hh, gg, fc_blocks, pps[2:], knn_flat)
           for f, hh, gg in zip(futs, fhs, fgs)]

    v0 = velocity_in[0, -1]
    outs = _decoder(xfs, v0, _mlp_t(params['decoder']))
    return outs[None]
